# Initial kernel scaffold; baseline (speedup 1.0000x reference)
import jax
import jax.numpy as jnp
from jax import lax
from jax.experimental import pallas as pl
from jax.experimental.pallas import tpu as pltpu
from jax.experimental.pallas import tpu_sc as plsc

N = 100000
B = 4
NB = 25000
DEG = 64
EB = NB * DEG
K_B = 2500
E_K = 40000

NPAD = 25008
NBP = 25024
SENT = 25008
CH = 2000
OUTP = 40960
NROW = 16

_CHUNKS = [(i * 2000, 2000) for i in range(12)] + [(24000, 1024)]


def _key_from_score(s):
  bits = lax.bitcast_convert_type(s, jnp.uint32)
  neg = (bits >> jnp.uint32(31)) != jnp.uint32(0)
  u = jnp.where(neg, ~bits, bits | jnp.uint32(0x80000000))
  return ~u


def _node_key(score_ref, v):
  s = plsc.load_gather(score_ref, [v])
  k = _key_from_score(s)
  return jnp.where(v < NB, k, jnp.uint32(0xFFFFFFFF))


def _add_from_shared(dst, src_sh, src_base, buf):
  for off, sz in _CHUNKS:
    pltpu.sync_copy(src_sh.at[pl.ds(src_base + off, sz)], buf.at[pl.ds(0, sz)])

    def addv(k2, _, off=off):
      dst[pl.ds(off + k2 * 16, 16)] = (
          dst[pl.ds(off + k2 * 16, 16)] + buf[pl.ds(k2 * 16, 16)])
      return 0

    lax.fori_loop(0, sz // 16, addv, 0)


def _zero(ref, nwords):
  def z(j, _):
    ref[pl.ds(j * 16, 16)] = jnp.zeros((16,), jnp.int32)
    return 0
  lax.fori_loop(0, nwords // 16, z, 0)


def _body(score_hbm, edge_hbm, batch_hbm, oute_hbm, outs_hbm,
          a1, a2, a3, hist, buf_r, buf_c, stg_p, stg_e, stg_s,
          reppk_sh, wc_sh, mgrid, oute_sh, outs_sh):
  del batch_hbm
  cid = lax.axis_index("c")
  sid = lax.axis_index("s")
  g = sid // 8
  t = sid % 8
  b = cid * 2 + g
  iota = lax.iota(jnp.int32, 16)

  pltpu.sync_copy(score_hbm.at[pl.ds(b * NB, NB)], a3.at[pl.ds(0, NB)])

  @pl.when(t == 0)
  def _phase1():
    def init(j, _):
      a1[pl.ds(j * 16, 16)] = j * 16 + iota
      return 0
    lax.fori_loop(0, NPAD // 16, init, 0)

    for p in range(4):
      src, dst = (a1, a2) if p % 2 == 0 else (a2, a1)
      sh = jnp.uint32(8 * p)
      _zero(hist, 256)

      def histo(j, _, src=src, sh=sh):
        v = src[pl.ds(j * 16, 16)]
        d = ((_node_key(a3, v) >> sh) & jnp.uint32(255)).astype(jnp.int32)
        cnt, is_last = plsc.scan_count(d)
        plsc.addupdate_scatter(hist, [d], cnt, mask=is_last)
        return 0
      lax.fori_loop(0, NPAD // 16, histo, 0)

      def excl(h, carry):
        vv = hist[pl.ds(h * 16, 16)]
        inc = plsc.cumsum(vv)
        hist[pl.ds(h * 16, 16)] = inc - vv + carry
        return carry + jnp.max(inc)
      lax.fori_loop(0, 16, excl, jnp.int32(0))

      def place(j, _, src=src, dst=dst, sh=sh):
        v = src[pl.ds(j * 16, 16)]
        d = ((_node_key(a3, v) >> sh) & jnp.uint32(255)).astype(jnp.int32)
        cnt, is_last = plsc.scan_count(d)
        base = plsc.load_gather(hist, [d])
        plsc.store_scatter(dst, [base + cnt - 1], v)
        plsc.store_scatter(hist, [d], base + cnt, mask=is_last)
        return 0
      lax.fori_loop(0, NPAD // 16, place, 0)

    def repb(j, carry_rep):
      v = a1[pl.ds(j * 16, 16)]
      vp = a1[pl.ds(jnp.maximum(j * 16 - 1, 0), 16)]
      k = _node_key(a3, v)
      kp = _node_key(a3, vp)
      neq = (k != kp) | ((j == 0) & (iota == 0))
      packed = jnp.where(neq, iota * 32768 + v, -1)
      packed = jnp.where((iota == 0) & jnp.logical_not(neq), carry_rep, packed)
      pm = plsc.cummax(packed)
      rep = pm & 32767
      selbit = jnp.where((j * 16 + iota) < K_B, jnp.int32(-2147483648),
                         jnp.int32(0))
      plsc.store_scatter(a2, [v], rep | selbit)
      return jnp.max(pm) & 32767
    lax.fori_loop(0, NPAD // 16, repb, jnp.int32(0))
    pltpu.sync_copy(a2, reppk_sh.at[pl.ds(g * NBP, NBP)])

  plsc.subcore_barrier()

  c0 = (t - 1) * 114 + jnp.minimum(t - 1, 2)
  nchunks = jnp.where(t <= 2, 115, 114)

  @pl.when(t > 0)
  def _phase2():
    pltpu.sync_copy(reppk_sh.at[pl.ds(g * NBP, NBP)], a1)
    _zero(a2, NBP)

    def chunk(ci, _):
      base_e = b * EB + ci * CH
      pltpu.sync_copy(edge_hbm.at[0, pl.ds(base_e, CH)], buf_r)
      pltpu.sync_copy(edge_hbm.at[1, pl.ds(base_e, CH)], buf_c)

      def vreg(j, _2):
        r = buf_r[pl.ds(j * 16, 16)] - b * NB
        c = buf_c[pl.ds(j * 16, 16)] - b * NB
        rp = plsc.load_gather(a1, [r])
        cp = plsc.load_gather(a1, [c])
        key = jnp.where(rp < 0, cp & 0x7FFFFFFF, SENT)
        cnt, is_last = plsc.scan_count(key)
        plsc.addupdate_scatter(a2, [key], cnt, mask=is_last)
        return 0
      lax.fori_loop(0, CH // 16, vreg, 0)
      return 0
    lax.fori_loop(c0, c0 + nchunks, chunk, 0)
    pltpu.sync_copy(a2, mgrid.at[pl.ds((g * 8 + t) * NBP, NBP)])

  plsc.subcore_barrier()

  @pl.when(t == 0)
  def _phase3_lead():
    pltpu.sync_copy(mgrid.at[pl.ds((g * 8 + 1) * NBP, NBP)], a2)

    def acc_tile(t2, _):
      _add_from_shared(a2, mgrid, (g * 8 + t2) * NBP, buf_r)
      return 0
    lax.fori_loop(2, 8, acc_tile, 0)

    for l in range(8):
      stg_p[0, pl.ds(l * 16, 16)] = jnp.full((16,), g * NBP + SENT + 1,
                                             jnp.int32)

    def wrow(jj, carry):
      def wvreg(l, carry2):
        carry_cum, carry_w = carry2
        j = jj * 8 + l
        v = a1[pl.ds(j * 16, 16)]
        vp = a1[pl.ds(jnp.maximum(j * 16 - 1, 0), 16)]
        k = _node_key(a3, v)
        kp = _node_key(a3, vp)
        neq = (k != kp) | ((j == 0) & (iota == 0))
        mv = plsc.load_gather(a2, [v])
        inc = plsc.cumsum(mv)
        excl2 = inc - mv + carry_cum
        w_in = jnp.where(neq, excl2, -1)
        w_in = jnp.where((iota == 0) & jnp.logical_not(neq), carry_w, w_in)
        wl = plsc.cummax(w_in)
        stg_p[0, pl.ds(l * 16, 16)] = g * NBP + v
        stg_e[0, pl.ds(l * 16, 16)] = wl
        return (carry_cum + jnp.max(inc), jnp.max(wl))
      nv = jnp.minimum(8, NPAD // 16 - jj * 8)
      carry = lax.fori_loop(0, nv, wvreg, carry)
      pltpu.sync_copy(stg_e.at[0], wc_sh.at[stg_p.at[0]])
      return carry
    lax.fori_loop(0, (NPAD // 16 + 7) // 8, wrow,
                  (jnp.int32(0), jnp.int32(0)))

    for l in range(8):
      stg_p[0, pl.ds(l * 16, 16)] = g * NBP + SENT + iota
      stg_e[0, pl.ds(l * 16, 16)] = jnp.full((16,), E_K, jnp.int32)
    pltpu.sync_copy(stg_e.at[0], wc_sh.at[stg_p.at[0]])

  @pl.when(t > 0)
  def _phase3a():
    _zero(a2, NBP)

    def acc_tile(t2, _):
      _add_from_shared(a2, mgrid, (g * 8 + t2) * NBP, buf_r)
      return 0
    lax.fori_loop(1, t, acc_tile, 0)

  plsc.subcore_barrier()

  @pl.when(t > 0)
  def _phase4():
    _add_from_shared(a2, wc_sh, g * NBP, buf_r)

    for l in range(5, 8):
      stg_p[15, pl.ds(l * 16, 16)] = g * OUTP + E_K + 512 + l * 16 + iota

    def chunk(ci, _):
      base_e = b * EB + ci * CH
      pltpu.sync_copy(edge_hbm.at[0, pl.ds(base_e, CH)], buf_r)
      pltpu.sync_copy(edge_hbm.at[1, pl.ds(base_e, CH)], buf_c)

      def row(jj, _2):
        def vreg(l, _3):
          j = jj * 8 + l
          r = buf_r[pl.ds(j * 16, 16)] - b * NB
          c = buf_c[pl.ds(j * 16, 16)] - b * NB
          rp = plsc.load_gather(a1, [r])
          cp = plsc.load_gather(a1, [c])
          key = jnp.where(rp < 0, cp & 0x7FFFFFFF, SENT)
          cnt, is_last = plsc.scan_count(key)
          base = plsc.load_gather(a2, [key])
          pos = base + cnt - 1
          plsc.store_scatter(a2, [key], base + cnt, mask=is_last)
          outpos = jnp.where(pos < E_K, pos, E_K + (pos & 511))
          stg_p[jj, pl.ds(l * 16, 16)] = g * OUTP + outpos
          stg_e[jj, pl.ds(l * 16, 16)] = base_e + j * 16 + iota
          stg_s[jj, pl.ds(l * 16, 16)] = plsc.load_gather(a3, [key])
          return 0
        nv = jnp.minimum(8, CH // 16 - jj * 8)
        lax.fori_loop(0, nv, vreg, 0)
        pltpu.sync_copy(stg_e.at[jj], oute_sh.at[stg_p.at[jj]])
        pltpu.sync_copy(stg_s.at[jj], outs_sh.at[stg_p.at[jj]])
        return 0
      lax.fori_loop(0, NROW, row, 0)
      return 0
    lax.fori_loop(c0, c0 + nchunks, chunk, 0)

  plsc.subcore_barrier()

  @pl.when(t == 0)
  def _final():
    pltpu.sync_copy(oute_sh.at[pl.ds(g * OUTP, E_K)],
                    oute_hbm.at[pl.ds(b * E_K, E_K)])
    pltpu.sync_copy(outs_sh.at[pl.ds(g * OUTP, E_K)],
                    outs_hbm.at[pl.ds(b * E_K, E_K)])


def kernel(score, edge_index, batch):
  mesh = plsc.VectorSubcoreMesh(core_axis_name="c", subcore_axis_name="s",
                                num_cores=2, num_subcores=16)
  f = pl.kernel(
      _body,
      out_type=(jax.ShapeDtypeStruct((B * E_K,), jnp.int32),
                jax.ShapeDtypeStruct((B * E_K,), jnp.float32)),
      mesh=mesh,
      scratch_types=[
          pltpu.VMEM((NBP,), jnp.int32),
          pltpu.VMEM((NBP,), jnp.int32),
          pltpu.VMEM((NBP,), jnp.float32),
          pltpu.VMEM((256,), jnp.int32),
          pltpu.VMEM((CH,), jnp.int32),
          pltpu.VMEM((CH,), jnp.int32),
          pltpu.VMEM((NROW, 128), jnp.int32),
          pltpu.VMEM((NROW, 128), jnp.int32),
          pltpu.VMEM((NROW, 128), jnp.float32),
          pltpu.VMEM_SHARED((2 * NBP,), jnp.int32),
          pltpu.VMEM_SHARED((2 * NBP,), jnp.int32),
          pltpu.VMEM_SHARED((16 * NBP,), jnp.int32),
          pltpu.VMEM_SHARED((2 * OUTP,), jnp.int32),
          pltpu.VMEM_SHARED((2 * OUTP,), jnp.float32),
      ],
  )
  return f(score, edge_index, batch)


# speedup vs baseline: 179.6266x; 179.6266x over previous
import jax
import jax.numpy as jnp
from jax import lax
from jax.experimental import pallas as pl
from jax.experimental.pallas import tpu as pltpu
from jax.experimental.pallas import tpu_sc as plsc

N = 100000
B = 4
NB = 25000
DEG = 64
EB = NB * DEG
K_B = 2500
E_K = 40000

NPAD = 25008
NBP = 25024
SENT = 25008
CH = 2000
OUTP = 40960
NROW = 16

_CHUNKS = [(i * 2000, 2000) for i in range(12)] + [(24000, 1024)]


def _key_from_score(s):
  bits = lax.bitcast_convert_type(s, jnp.uint32)
  neg = (bits >> jnp.uint32(31)) != jnp.uint32(0)
  u = jnp.where(neg, ~bits, bits | jnp.uint32(0x80000000))
  return ~u


def _node_key(score_ref, v):
  s = plsc.load_gather(score_ref, [v])
  k = _key_from_score(s)
  return jnp.where(v < NB, k, jnp.uint32(0xFFFFFFFF))


def _add_from_shared(dst, src_sh, src_base, buf):
  for off, sz in _CHUNKS:
    pltpu.sync_copy(src_sh.at[pl.ds(src_base + off, sz)], buf.at[pl.ds(0, sz)])

    def addv(k2, _, off=off):
      dst[pl.ds(off + k2 * 16, 16)] = (
          dst[pl.ds(off + k2 * 16, 16)] + buf[pl.ds(k2 * 16, 16)])
      return 0

    lax.fori_loop(0, sz // 16, addv, 0)


def _zero(ref, nwords):
  def z(j, _):
    ref[pl.ds(j * 16, 16)] = jnp.zeros((16,), jnp.int32)
    return 0
  lax.fori_loop(0, nwords // 16, z, 0)


def _body(score_hbm, row_hbm, col_hbm, batch_hbm, oute_hbm, outs_hbm,
          a1, a2, a3, hist, buf_r, buf_c, buf_s, stg_p, stg_e, stg_s,
          reppk_sh, wc_sh, mgrid, oute_sh, outs_sh):
  del batch_hbm
  cid = lax.axis_index("c")
  sid = lax.axis_index("s")
  g = sid // 8
  t = sid % 8
  b = cid * 2 + g
  iota = lax.iota(jnp.int32, 16)

  pltpu.sync_copy(score_hbm.at[pl.ds(b * NB, NB)], a3.at[pl.ds(0, NB)])

  @pl.when(t == 0)
  def _phase1():
    def init(j, _):
      a1[pl.ds(j * 16, 16)] = j * 16 + iota
      return 0
    lax.fori_loop(0, NPAD // 16, init, 0)

    for p in range(4):
      src, dst = (a1, a2) if p % 2 == 0 else (a2, a1)
      sh = jnp.uint32(8 * p)
      _zero(hist, 256)

      def histo(j, _, src=src, sh=sh):
        v = src[pl.ds(j * 16, 16)]
        d = ((_node_key(a3, v) >> sh) & jnp.uint32(255)).astype(jnp.int32)
        cnt, is_last = plsc.scan_count(d)
        plsc.addupdate_scatter(hist, [d], cnt, mask=is_last)
        return 0
      lax.fori_loop(0, NPAD // 16, histo, 0)

      def excl(h, carry):
        vv = hist[pl.ds(h * 16, 16)]
        inc = plsc.cumsum(vv)
        hist[pl.ds(h * 16, 16)] = inc - vv + carry
        return carry + jnp.max(inc)
      lax.fori_loop(0, 16, excl, jnp.int32(0))

      def place(j, _, src=src, dst=dst, sh=sh):
        v = src[pl.ds(j * 16, 16)]
        d = ((_node_key(a3, v) >> sh) & jnp.uint32(255)).astype(jnp.int32)
        cnt, is_last = plsc.scan_count(d)
        base = plsc.load_gather(hist, [d])
        plsc.store_scatter(dst, [base + cnt - 1], v)
        plsc.store_scatter(hist, [d], base + cnt, mask=is_last)
        return 0
      lax.fori_loop(0, NPAD // 16, place, 0)

    def repb(j, carry_rep):
      v = a1[pl.ds(j * 16, 16)]
      vp = a1[pl.ds(jnp.maximum(j * 16 - 1, 0), 16)]
      k = _node_key(a3, v)
      kp = _node_key(a3, vp)
      neq = (k != kp) | (j == 0)
      packed = jnp.where(neq, iota * 32768 + v, -1)
      packed = jnp.where((iota == 0) & jnp.logical_not(neq), carry_rep, packed)
      pm = plsc.cummax(packed)
      rep = pm & 32767
      selbit = jnp.where((j * 16 + iota) < K_B, jnp.int32(-2147483648),
                         jnp.int32(0))
      plsc.store_scatter(a2, [v], rep | selbit)
      return jnp.max(pm) & 32767
    lax.fori_loop(0, NPAD // 16, repb, jnp.int32(0))
    pltpu.sync_copy(a2, reppk_sh.at[pl.ds(g * NBP, NBP)])

  plsc.subcore_barrier()

  c0 = (t - 1) * 114 + jnp.minimum(t - 1, 2)
  nchunks = jnp.where(t <= 2, 115, 114)

  @pl.when(t > 0)
  def _phase2():
    pltpu.sync_copy(reppk_sh.at[pl.ds(g * NBP, NBP)], a1)
    _zero(a2, NBP)

    def chunk(ci, _):
      base_e = b * EB + ci * CH
      pltpu.sync_copy(row_hbm.at[pl.ds(base_e, CH)], buf_r)
      pltpu.sync_copy(col_hbm.at[pl.ds(base_e, CH)], buf_c)

      def vreg(j, _2):
        r = buf_r[pl.ds(j * 16, 16)] - b * NB
        c = buf_c[pl.ds(j * 16, 16)] - b * NB
        rp = plsc.load_gather(a1, [r])
        cp = plsc.load_gather(a1, [c])
        key = jnp.where(rp < 0, cp & 0x7FFFFFFF, SENT)
        cnt, is_last = plsc.scan_count(key)
        plsc.addupdate_scatter(a2, [key], cnt, mask=is_last)
        return 0
      lax.fori_loop(0, CH // 16, vreg, 0)
      return 0
    lax.fori_loop(c0, c0 + nchunks, chunk, 0)
    pltpu.sync_copy(a2, mgrid.at[pl.ds((g * 8 + t) * NBP, NBP)])

  plsc.subcore_barrier()

  @pl.when(t == 0)
  def _phase3_lead():
    pltpu.sync_copy(mgrid.at[pl.ds((g * 8 + 1) * NBP, NBP)], a2)

    def acc_tile(t2, _):
      _add_from_shared(a2, mgrid, (g * 8 + t2) * NBP, buf_r)
      return 0
    lax.fori_loop(2, 8, acc_tile, 0)

    for l in range(8):
      stg_p[0, pl.ds(l * 16, 16)] = jnp.full((16,), g * NBP + SENT + 1,
                                             jnp.int32)

    def wrow(jj, carry):
      def wvreg(l, carry2):
        carry_cum, carry_w = carry2
        j = jj * 8 + l
        v = a1[pl.ds(j * 16, 16)]
        vp = a1[pl.ds(jnp.maximum(j * 16 - 1, 0), 16)]
        k = _node_key(a3, v)
        kp = _node_key(a3, vp)
        neq = (k != kp) | (j == 0)
        mv = plsc.load_gather(a2, [v])
        inc = plsc.cumsum(mv)
        excl2 = inc - mv + carry_cum
        w_in = jnp.where(neq, excl2, -1)
        w_in = jnp.where((iota == 0) & jnp.logical_not(neq), carry_w, w_in)
        wl = plsc.cummax(w_in)
        stg_p[0, pl.ds(l * 16, 16)] = g * NBP + v
        stg_e[0, pl.ds(l * 16, 16)] = wl
        return (carry_cum + jnp.max(inc), jnp.max(wl))
      nv = jnp.minimum(8, NPAD // 16 - jj * 8)
      carry = lax.fori_loop(0, nv, wvreg, carry)
      pltpu.sync_copy(stg_e.at[0], wc_sh.at[stg_p.at[0]])
      return carry
    lax.fori_loop(0, (NPAD // 16 + 7) // 8, wrow,
                  (jnp.int32(0), jnp.int32(0)))

    for l in range(8):
      stg_p[0, pl.ds(l * 16, 16)] = g * NBP + SENT + iota
      stg_e[0, pl.ds(l * 16, 16)] = jnp.full((16,), E_K, jnp.int32)
    pltpu.sync_copy(stg_e.at[0], wc_sh.at[stg_p.at[0]])

  @pl.when(t > 0)
  def _phase3a():
    _zero(a2, NBP)

    def acc_tile(t2, _):
      _add_from_shared(a2, mgrid, (g * 8 + t2) * NBP, buf_r)
      return 0
    lax.fori_loop(1, t, acc_tile, 0)

  plsc.subcore_barrier()

  @pl.when(t > 0)
  def _phase4():
    _add_from_shared(a2, wc_sh, g * NBP, buf_r)

    for l in range(5, 8):
      stg_p[15, pl.ds(l * 16, 16)] = g * OUTP + E_K + 512 + l * 16 + iota

    def chunk(ci, _):
      base_e = b * EB + ci * CH
      pltpu.sync_copy(row_hbm.at[pl.ds(base_e, CH)], buf_r)
      pltpu.sync_copy(col_hbm.at[pl.ds(base_e, CH)], buf_c)

      def row(jj, _2):
        def vreg(l, _3):
          j = jj * 8 + l
          r = buf_r[pl.ds(j * 16, 16)] - b * NB
          c = buf_c[pl.ds(j * 16, 16)] - b * NB
          rp = plsc.load_gather(a1, [r])
          cp = plsc.load_gather(a1, [c])
          key = jnp.where(rp < 0, cp & 0x7FFFFFFF, SENT)
          cnt, is_last = plsc.scan_count(key)
          base = plsc.load_gather(a2, [key])
          pos = base + cnt - 1
          plsc.store_scatter(a2, [key], base + cnt, mask=is_last)
          outpos = jnp.where(pos < E_K, pos, E_K + (pos & 511))
          stg_p[jj, pl.ds(l * 16, 16)] = g * OUTP + outpos
          stg_e[jj, pl.ds(l * 16, 16)] = base_e + j * 16 + iota
          stg_s[jj, pl.ds(l * 16, 16)] = plsc.load_gather(a3, [key])
          return 0
        nv = jnp.minimum(8, CH // 16 - jj * 8)
        lax.fori_loop(0, nv, vreg, 0)
        pltpu.sync_copy(stg_e.at[jj], oute_sh.at[stg_p.at[jj]])
        pltpu.sync_copy(stg_s.at[jj], outs_sh.at[stg_p.at[jj]])
        return 0
      lax.fori_loop(0, NROW, row, 0)
      return 0
    lax.fori_loop(c0, c0 + nchunks, chunk, 0)

  plsc.subcore_barrier()

  @pl.when(t == 0)
  def _final():
    def out_chunk(k2, _):
      pltpu.sync_copy(oute_sh.at[pl.ds(g * OUTP + k2 * CH, CH)], buf_r)
      pltpu.sync_copy(buf_r, oute_hbm.at[pl.ds(b * E_K + k2 * CH, CH)])
      pltpu.sync_copy(outs_sh.at[pl.ds(g * OUTP + k2 * CH, CH)], buf_s)
      pltpu.sync_copy(buf_s, outs_hbm.at[pl.ds(b * E_K + k2 * CH, CH)])
      return 0
    lax.fori_loop(0, E_K // CH, out_chunk, 0)


def kernel(score, edge_index, batch):
  mesh = plsc.VectorSubcoreMesh(core_axis_name="c", subcore_axis_name="s",
                                num_cores=2, num_subcores=16)
  f = pl.kernel(
      _body,
      compiler_params=pltpu.CompilerParams(needs_layout_passes=False),
      out_type=(jax.ShapeDtypeStruct((B * E_K,), jnp.int32),
                jax.ShapeDtypeStruct((B * E_K,), jnp.float32)),
      mesh=mesh,
      scratch_types=[
          pltpu.VMEM((NBP,), jnp.int32),
          pltpu.VMEM((NBP,), jnp.int32),
          pltpu.VMEM((NBP,), jnp.float32),
          pltpu.VMEM((256,), jnp.int32),
          pltpu.VMEM((CH,), jnp.int32),
          pltpu.VMEM((CH,), jnp.int32),
          pltpu.VMEM((CH,), jnp.float32),
          pltpu.VMEM((NROW, 128), jnp.int32),
          pltpu.VMEM((NROW, 128), jnp.int32),
          pltpu.VMEM((NROW, 128), jnp.float32),
          pltpu.VMEM_SHARED((2 * NBP,), jnp.int32),
          pltpu.VMEM_SHARED((2 * NBP,), jnp.int32),
          pltpu.VMEM_SHARED((16 * NBP,), jnp.int32),
          pltpu.VMEM_SHARED((2 * OUTP,), jnp.int32),
          pltpu.VMEM_SHARED((2 * OUTP,), jnp.float32),
      ],
  )
  return f(score, edge_index[0], edge_index[1], batch)


# speedup vs baseline: 255.1136x; 1.4202x over previous
import jax
import jax.numpy as jnp
from jax import lax
from jax.experimental import pallas as pl
from jax.experimental.pallas import tpu as pltpu
from jax.experimental.pallas import tpu_sc as plsc

N = 100000
B = 4
NB = 25000
DEG = 64
EB = NB * DEG
K_B = 2500
E_K = 40000

NPAD = 25008
NBP = 25024
SENT = 25008
CH = 1280
NCH = EB // CH
OUTP = 40960
NRING = 8
FCH = 2000

_CHUNKS = [(i * 2000, 2000) for i in range(12)] + [(24000, 1024)]


def _key_from_score(s):
  bits = lax.bitcast_convert_type(s, jnp.uint32)
  neg = (bits >> jnp.uint32(31)) != jnp.uint32(0)
  u = jnp.where(neg, ~bits, bits | jnp.uint32(0x80000000))
  return ~u


def _node_key(score_ref, v):
  s = plsc.load_gather(score_ref, [v])
  k = _key_from_score(s)
  return jnp.where(v < NB, k, jnp.uint32(0xFFFFFFFF))


def _add_from_shared(dst, src_sh, src_base, buf):
  for off, sz in _CHUNKS:
    pltpu.sync_copy(src_sh.at[pl.ds(src_base + off, sz)],
                    buf.at[pl.ds(0, sz)])

    def addv(k2, _, off=off):
      dst[pl.ds(off + k2 * 16, 16)] = (
          dst[pl.ds(off + k2 * 16, 16)] + buf[pl.ds(k2 * 16, 16)])
      return 0

    lax.fori_loop(0, sz // 16, addv, 0)


def _zero(ref, nwords):
  def z(j, _):
    ref[pl.ds(j * 16, 16)] = jnp.zeros((16,), jnp.int32)
    return 0
  lax.fori_loop(0, nwords // 16, z, 0)


def _body(score_hbm, row_hbm, col_hbm, batch_hbm, oute_hbm, outs_hbm,
          a1, a2, a3, hist, buf_s, stg_p, stg_e, stg_s,
          ebuf_r, ebuf_c, sem_in, sem_out,
          reppk_sh, wc_sh, mgrid, oute_sh, outs_sh):
  del batch_hbm
  cid = lax.axis_index("c")
  sid = lax.axis_index("s")
  g = sid // 8
  t = sid % 8
  b = cid * 2 + g
  iota = lax.iota(jnp.int32, 16)

  pltpu.sync_copy(score_hbm.at[pl.ds(b * NB, NB)], a3.at[pl.ds(0, NB)])

  @pl.when(t == 0)
  def _phase1():
    def init(j, _):
      a1[pl.ds(j * 16, 16)] = j * 16 + iota
      return 0
    lax.fori_loop(0, NPAD // 16, init, 0)

    for p in range(4):
      src, dst = (a1, a2) if p % 2 == 0 else (a2, a1)
      sh = jnp.uint32(8 * p)
      _zero(hist, 256)

      def histo(j, _, src=src, sh=sh):
        v = src[pl.ds(j * 16, 16)]
        d = ((_node_key(a3, v) >> sh) & jnp.uint32(255)).astype(jnp.int32)
        cnt, is_last = plsc.scan_count(d)
        plsc.addupdate_scatter(hist, [d], cnt, mask=is_last)
        return 0
      lax.fori_loop(0, NPAD // 16, histo, 0)

      def excl(h, carry):
        vv = hist[pl.ds(h * 16, 16)]
        inc = plsc.cumsum(vv)
        hist[pl.ds(h * 16, 16)] = inc - vv + carry
        return carry + jnp.max(inc)
      lax.fori_loop(0, 16, excl, jnp.int32(0))

      def place(j, _, src=src, dst=dst, sh=sh):
        v = src[pl.ds(j * 16, 16)]
        d = ((_node_key(a3, v) >> sh) & jnp.uint32(255)).astype(jnp.int32)
        cnt, is_last = plsc.scan_count(d)
        base = plsc.load_gather(hist, [d])
        plsc.store_scatter(dst, [base + cnt - 1], v)
        plsc.store_scatter(hist, [d], base + cnt, mask=is_last)
        return 0
      lax.fori_loop(0, NPAD // 16, place, 0)

    def repb(j, carry_rep):
      v = a1[pl.ds(j * 16, 16)]
      vp = a1[pl.ds(jnp.maximum(j * 16 - 1, 0), 16)]
      k = _node_key(a3, v)
      kp = _node_key(a3, vp)
      neq = (k != kp) | (j == 0)
      packed = jnp.where(neq, iota * 32768 + v, -1)
      packed = jnp.where((iota == 0) & jnp.logical_not(neq), carry_rep, packed)
      pm = plsc.cummax(packed)
      rep = pm & 32767
      selbit = jnp.where((j * 16 + iota) < K_B, jnp.int32(-2147483648),
                         jnp.int32(0))
      plsc.store_scatter(a2, [v], rep | selbit)
      return jnp.max(pm) & 32767
    lax.fori_loop(0, NPAD // 16, repb, jnp.int32(0))
    pltpu.sync_copy(a2, reppk_sh.at[pl.ds(g * NBP, NBP)])

  plsc.subcore_barrier()

  c0 = (t - 1) * 178 + jnp.minimum(t - 1, 4)
  nchunks = jnp.where(t <= 4, 179, 178)

  @pl.when(t > 0)
  def _phase2():
    pltpu.sync_copy(reppk_sh.at[pl.ds(g * NBP, NBP)], a1)
    _zero(a2, NBP)

    pltpu.async_copy(row_hbm.at[pl.ds(b * EB + c0 * CH, CH)],
                     ebuf_r.at[pl.ds(0, CH)], sem_in)
    pltpu.async_copy(col_hbm.at[pl.ds(b * EB + c0 * CH, CH)],
                     ebuf_c.at[pl.ds(0, CH)], sem_in)

    def chunk(ci, _):
      par = lax.rem(ci - c0, 2)
      base_e = b * EB + ci * CH
      pltpu.make_async_copy(row_hbm.at[pl.ds(base_e, CH)],
                            ebuf_r.at[pl.ds(par * CH, CH)], sem_in).wait()
      pltpu.make_async_copy(col_hbm.at[pl.ds(base_e, CH)],
                            ebuf_c.at[pl.ds(par * CH, CH)], sem_in).wait()

      @pl.when(ci + 1 < c0 + nchunks)
      def _pref():
        nbase = b * EB + (ci + 1) * CH
        pltpu.async_copy(row_hbm.at[pl.ds(nbase, CH)],
                         ebuf_r.at[pl.ds((1 - par) * CH, CH)], sem_in)
        pltpu.async_copy(col_hbm.at[pl.ds(nbase, CH)],
                         ebuf_c.at[pl.ds((1 - par) * CH, CH)], sem_in)

      def vreg(j, _2):
        r = ebuf_r[pl.ds(par * CH + j * 16, 16)] - b * NB
        c = ebuf_c[pl.ds(par * CH + j * 16, 16)] - b * NB
        rp = plsc.load_gather(a1, [r])
        cp = plsc.load_gather(a1, [c])
        key = jnp.where(rp < 0, cp & 0x7FFFFFFF, SENT)
        cnt, is_last = plsc.scan_count(key)
        plsc.addupdate_scatter(a2, [key], cnt, mask=is_last)
        return 0
      lax.fori_loop(0, CH // 16, vreg, 0)
      return 0
    lax.fori_loop(c0, c0 + nchunks, chunk, 0)
    pltpu.sync_copy(a2, mgrid.at[pl.ds((g * 7 + t - 1) * NBP, NBP)])

  plsc.subcore_barrier()

  @pl.when(t == 0)
  def _phase3_lead():
    pltpu.sync_copy(mgrid.at[pl.ds(g * 7 * NBP, NBP)], a2)

    def acc_tile(t2, _):
      _add_from_shared(a2, mgrid, (g * 7 + t2 - 1) * NBP, ebuf_r)
      return 0
    lax.fori_loop(2, 8, acc_tile, 0)

    for l in range(8):
      stg_p[0, pl.ds(l * 16, 16)] = jnp.full((16,), g * NBP + SENT + 1,
                                             jnp.int32)

    def wrow(jj, carry):
      def wvreg(l, carry2):
        carry_cum, carry_w = carry2
        j = jj * 8 + l
        v = a1[pl.ds(j * 16, 16)]
        vp = a1[pl.ds(jnp.maximum(j * 16 - 1, 0), 16)]
        k = _node_key(a3, v)
        kp = _node_key(a3, vp)
        neq = (k != kp) | (j == 0)
        mv = plsc.load_gather(a2, [v])
        inc = plsc.cumsum(mv)
        excl2 = inc - mv + carry_cum
        w_in = jnp.where(neq, excl2, -1)
        w_in = jnp.where((iota == 0) & jnp.logical_not(neq), carry_w, w_in)
        wl = plsc.cummax(w_in)
        stg_p[0, pl.ds(l * 16, 16)] = g * NBP + v
        stg_e[0, pl.ds(l * 16, 16)] = wl
        return (carry_cum + jnp.max(inc), jnp.max(wl))
      nv = jnp.minimum(8, NPAD // 16 - jj * 8)
      carry = lax.fori_loop(0, nv, wvreg, carry)
      pltpu.sync_copy(stg_e.at[0], wc_sh.at[stg_p.at[0]])
      return carry
    lax.fori_loop(0, (NPAD // 16 + 7) // 8, wrow,
                  (jnp.int32(0), jnp.int32(0)))

    for l in range(8):
      stg_p[0, pl.ds(l * 16, 16)] = g * NBP + SENT + iota
      stg_e[0, pl.ds(l * 16, 16)] = jnp.full((16,), E_K, jnp.int32)
    pltpu.sync_copy(stg_e.at[0], wc_sh.at[stg_p.at[0]])

  @pl.when(t > 0)
  def _phase3a():
    _zero(a2, NBP)

    def acc_tile(t2, _):
      _add_from_shared(a2, mgrid, (g * 7 + t2 - 1) * NBP, ebuf_r)
      return 0
    lax.fori_loop(1, t, acc_tile, 0)

  plsc.subcore_barrier()

  @pl.when(t > 0)
  def _phase4():
    _add_from_shared(a2, wc_sh, g * NBP, ebuf_r)

    pltpu.async_copy(row_hbm.at[pl.ds(b * EB + c0 * CH, CH)],
                     ebuf_r.at[pl.ds(0, CH)], sem_in)
    pltpu.async_copy(col_hbm.at[pl.ds(b * EB + c0 * CH, CH)],
                     ebuf_c.at[pl.ds(0, CH)], sem_in)

    def drain_row(rr):
      pltpu.make_async_copy(stg_e.at[rr], oute_sh.at[stg_p.at[rr]],
                            sem_out).wait()
      pltpu.make_async_copy(stg_s.at[rr], outs_sh.at[stg_p.at[rr]],
                            sem_out).wait()

    def chunk(ci, _):
      par = lax.rem(ci - c0, 2)
      base_e = b * EB + ci * CH
      pltpu.make_async_copy(row_hbm.at[pl.ds(base_e, CH)],
                            ebuf_r.at[pl.ds(par * CH, CH)], sem_in).wait()
      pltpu.make_async_copy(col_hbm.at[pl.ds(base_e, CH)],
                            ebuf_c.at[pl.ds(par * CH, CH)], sem_in).wait()

      @pl.when(ci + 1 < c0 + nchunks)
      def _pref():
        nbase = b * EB + (ci + 1) * CH
        pltpu.async_copy(row_hbm.at[pl.ds(nbase, CH)],
                         ebuf_r.at[pl.ds((1 - par) * CH, CH)], sem_in)
        pltpu.async_copy(col_hbm.at[pl.ds(nbase, CH)],
                         ebuf_c.at[pl.ds((1 - par) * CH, CH)], sem_in)

      def row(jj, _2):
        rr = lax.rem(jj, NRING)

        def vreg(l, _3):
          j = jj * 8 + l
          r = ebuf_r[pl.ds(par * CH + j * 16, 16)] - b * NB
          c = ebuf_c[pl.ds(par * CH + j * 16, 16)] - b * NB
          rp = plsc.load_gather(a1, [r])
          cp = plsc.load_gather(a1, [c])
          key = jnp.where(rp < 0, cp & 0x7FFFFFFF, SENT)
          cnt, is_last = plsc.scan_count(key)
          base = plsc.load_gather(a2, [key])
          pos = base + cnt - 1
          plsc.store_scatter(a2, [key], base + cnt, mask=is_last)
          outpos = jnp.where(pos < E_K, pos, E_K + (pos & 511))
          stg_p[rr, pl.ds(l * 16, 16)] = g * OUTP + outpos
          stg_e[rr, pl.ds(l * 16, 16)] = base_e + j * 16 + iota
          stg_s[rr, pl.ds(l * 16, 16)] = plsc.load_gather(a3, [key])
          return 0
        lax.fori_loop(0, 8, vreg, 0)
        pltpu.async_copy(stg_e.at[rr], oute_sh.at[stg_p.at[rr]], sem_out)
        pltpu.async_copy(stg_s.at[rr], outs_sh.at[stg_p.at[rr]], sem_out)

        @pl.when(jj >= 4)
        def _ringdrain():
          drain_row(lax.rem(jj - 4, NRING))
        return 0
      lax.fori_loop(0, CH // 128, row, 0)

      def tail_drain(jj, _2):
        drain_row(lax.rem(jj, NRING))
        return 0
      lax.fori_loop(CH // 128 - 4, CH // 128, tail_drain, 0)
      return 0
    lax.fori_loop(c0, c0 + nchunks, chunk, 0)

  plsc.subcore_barrier()

  @pl.when(t == 0)
  def _final():
    def out_chunk(k2, _):
      pltpu.sync_copy(oute_sh.at[pl.ds(g * OUTP + k2 * FCH, FCH)],
                      ebuf_r.at[pl.ds(0, FCH)])
      pltpu.sync_copy(ebuf_r.at[pl.ds(0, FCH)],
                      oute_hbm.at[pl.ds(b * E_K + k2 * FCH, FCH)])
      pltpu.sync_copy(outs_sh.at[pl.ds(g * OUTP + k2 * FCH, FCH)], buf_s)
      pltpu.sync_copy(buf_s, outs_hbm.at[pl.ds(b * E_K + k2 * FCH, FCH)])
      return 0
    lax.fori_loop(0, E_K // FCH, out_chunk, 0)


def kernel(score, edge_index, batch):
  mesh = plsc.VectorSubcoreMesh(core_axis_name="c", subcore_axis_name="s",
                                num_cores=2, num_subcores=16)
  f = pl.kernel(
      _body,
      compiler_params=pltpu.CompilerParams(needs_layout_passes=False),
      out_type=(jax.ShapeDtypeStruct((B * E_K,), jnp.int32),
                jax.ShapeDtypeStruct((B * E_K,), jnp.float32)),
      mesh=mesh,
      scratch_types=[
          pltpu.VMEM((NBP,), jnp.int32),
          pltpu.VMEM((NBP,), jnp.int32),
          pltpu.VMEM((NBP,), jnp.float32),
          pltpu.VMEM((256,), jnp.int32),
          pltpu.VMEM((FCH,), jnp.float32),
          pltpu.VMEM((NRING, 128), jnp.int32),
          pltpu.VMEM((NRING, 128), jnp.int32),
          pltpu.VMEM((NRING, 128), jnp.float32),
          pltpu.VMEM((2 * CH,), jnp.int32),
          pltpu.VMEM((2 * CH,), jnp.int32),
          pltpu.SemaphoreType.DMA,
          pltpu.SemaphoreType.DMA,
          pltpu.VMEM_SHARED((2 * NBP,), jnp.int32),
          pltpu.VMEM_SHARED((2 * NBP,), jnp.int32),
          pltpu.VMEM_SHARED((14 * NBP,), jnp.int32),
          pltpu.VMEM_SHARED((2 * OUTP,), jnp.int32),
          pltpu.VMEM_SHARED((2 * OUTP,), jnp.float32),
      ],
  )
  return f(score, edge_index[0], edge_index[1], batch)


# speedup vs baseline: 351.5320x; 1.3779x over previous
import jax
import jax.numpy as jnp
from jax import lax
from jax.experimental import pallas as pl
from jax.experimental.pallas import tpu as pltpu
from jax.experimental.pallas import tpu_sc as plsc

N = 100000
B = 4
NB = 25000
DEG = 64
EB = NB * DEG
K_B = 2500
E_K = 40000

NPAD = 25008
NBP = 25024
SENT = 25008
CH = 1280
NCH = EB // CH
OUTP = 40960
NRING = 8
FCH = 2000

_CHUNKS = [(i * 2000, 2000) for i in range(12)] + [(24000, 1024)]


def _key_from_score(s):
  bits = lax.bitcast_convert_type(s, jnp.uint32)
  neg = (bits >> jnp.uint32(31)) != jnp.uint32(0)
  u = jnp.where(neg, ~bits, bits | jnp.uint32(0x80000000))
  return ~u


def _node_key(score_ref, v):
  s = plsc.load_gather(score_ref, [v])
  k = _key_from_score(s)
  return jnp.where(v < NB, k, jnp.uint32(0xFFFFFFFF))


def _add_from_shared(dst, src_sh, src_base, buf):
  for off, sz in _CHUNKS:
    pltpu.sync_copy(src_sh.at[pl.ds(src_base + off, sz)],
                    buf.at[pl.ds(0, sz)])

    def addv(k2, _, off=off):
      dst[pl.ds(off + k2 * 16, 16)] = (
          dst[pl.ds(off + k2 * 16, 16)] + buf[pl.ds(k2 * 16, 16)])
      return 0

    lax.fori_loop(0, sz // 16, addv, 0)


def _zero(ref, nwords):
  def z(j, _):
    ref[pl.ds(j * 16, 16)] = jnp.zeros((16,), jnp.int32)
    return 0
  lax.fori_loop(0, nwords // 16, z, 0)


def _body(score_hbm, row_hbm, col_hbm, batch_hbm, oute_hbm, outs_hbm,
          a1, a2, a3, hist, buf_s, stg_p, stg_e, stg_s,
          ebuf_r, ebuf_c, sem_in, sem_out,
          reppk_sh, wc_sh, mgrid, oute_sh, outs_sh):
  del batch_hbm
  cid = lax.axis_index("c")
  sid = lax.axis_index("s")
  g = sid // 8
  t = sid % 8
  b = cid * 2 + g
  iota = lax.iota(jnp.int32, 16)

  pltpu.sync_copy(score_hbm.at[pl.ds(b * NB, NB)], a3.at[pl.ds(0, NB)])

  @pl.when(t == 0)
  def _phase1():
    def init(j, _):
      a1[pl.ds(j * 16, 16)] = j * 16 + iota
      return 0
    lax.fori_loop(0, NPAD // 16, init, 0)

    for p in range(4):
      src, dst = (a1, a2) if p % 2 == 0 else (a2, a1)
      sh = jnp.uint32(8 * p)
      _zero(hist, 256)

      @plsc.parallel_loop(0, NPAD // 16, unroll=4)
      def histo(j, src=src, sh=sh):
        v = src[pl.ds(j * 16, 16)]
        d = ((_node_key(a3, v) >> sh) & jnp.uint32(255)).astype(jnp.int32)
        cnt, is_last = plsc.scan_count(d)
        plsc.addupdate_scatter(hist, [d], cnt, mask=is_last)

      def excl(h, carry):
        vv = hist[pl.ds(h * 16, 16)]
        inc = plsc.cumsum(vv)
        hist[pl.ds(h * 16, 16)] = inc - vv + carry
        return carry + jnp.max(inc)
      lax.fori_loop(0, 16, excl, jnp.int32(0))

      def place(j, _, src=src, dst=dst, sh=sh):
        v = src[pl.ds(j * 16, 16)]
        d = ((_node_key(a3, v) >> sh) & jnp.uint32(255)).astype(jnp.int32)
        cnt, is_last = plsc.scan_count(d)
        base = plsc.load_gather(hist, [d])
        plsc.store_scatter(dst, [base + cnt - 1], v)
        plsc.store_scatter(hist, [d], base + cnt, mask=is_last)
        return 0
      lax.fori_loop(0, NPAD // 16, place, 0)

    def repb(j, carry_rep):
      v = a1[pl.ds(j * 16, 16)]
      vp = a1[pl.ds(jnp.maximum(j * 16 - 1, 0), 16)]
      k = _node_key(a3, v)
      kp = _node_key(a3, vp)
      neq = (k != kp) | (j == 0)
      packed = jnp.where(neq, iota * 32768 + v, -1)
      packed = jnp.where((iota == 0) & jnp.logical_not(neq), carry_rep, packed)
      pm = plsc.cummax(packed)
      rep = pm & 32767
      selbit = jnp.where((j * 16 + iota) < K_B, jnp.int32(-2147483648),
                         jnp.int32(0))
      plsc.store_scatter(a2, [v], rep | selbit)
      return jnp.max(pm) & 32767
    lax.fori_loop(0, NPAD // 16, repb, jnp.int32(0))
    pltpu.sync_copy(a2, reppk_sh.at[pl.ds(g * NBP, NBP)])

  plsc.subcore_barrier()

  c0 = (t - 1) * 178 + jnp.minimum(t - 1, 4)
  nchunks = jnp.where(t <= 4, 179, 178)

  @pl.when(t > 0)
  def _phase2():
    pltpu.sync_copy(reppk_sh.at[pl.ds(g * NBP, NBP)], a1)
    _zero(a2, NBP)

    pltpu.async_copy(row_hbm.at[pl.ds(b * EB + c0 * CH, CH)],
                     ebuf_r.at[pl.ds(0, CH)], sem_in)
    pltpu.async_copy(col_hbm.at[pl.ds(b * EB + c0 * CH, CH)],
                     ebuf_c.at[pl.ds(0, CH)], sem_in)

    def chunk(ci, _):
      par = lax.rem(ci - c0, 2)
      base_e = b * EB + ci * CH
      pltpu.make_async_copy(row_hbm.at[pl.ds(base_e, CH)],
                            ebuf_r.at[pl.ds(par * CH, CH)], sem_in).wait()
      pltpu.make_async_copy(col_hbm.at[pl.ds(base_e, CH)],
                            ebuf_c.at[pl.ds(par * CH, CH)], sem_in).wait()

      @pl.when(ci + 1 < c0 + nchunks)
      def _pref():
        nbase = b * EB + (ci + 1) * CH
        pltpu.async_copy(row_hbm.at[pl.ds(nbase, CH)],
                         ebuf_r.at[pl.ds((1 - par) * CH, CH)], sem_in)
        pltpu.async_copy(col_hbm.at[pl.ds(nbase, CH)],
                         ebuf_c.at[pl.ds((1 - par) * CH, CH)], sem_in)

      @plsc.parallel_loop(0, CH // 16, unroll=4)
      def vreg(j):
        r = ebuf_r[pl.ds(par * CH + j * 16, 16)] - b * NB
        c = ebuf_c[pl.ds(par * CH + j * 16, 16)] - b * NB
        rp = plsc.load_gather(a1, [r])
        cp = plsc.load_gather(a1, [c])
        key = jnp.where(rp < 0, cp & 0x7FFFFFFF, SENT)
        cnt, is_last = plsc.scan_count(key)
        plsc.addupdate_scatter(a2, [key], cnt, mask=is_last)
      return 0
    lax.fori_loop(c0, c0 + nchunks, chunk, 0)
    pltpu.sync_copy(a2, mgrid.at[pl.ds((g * 7 + t - 1) * NBP, NBP)])

  plsc.subcore_barrier()

  @pl.when(t == 0)
  def _phase3_lead():
    pltpu.sync_copy(mgrid.at[pl.ds(g * 7 * NBP, NBP)], a2)

    def acc_tile(t2, _):
      _add_from_shared(a2, mgrid, (g * 7 + t2 - 1) * NBP, ebuf_r)
      return 0
    lax.fori_loop(2, 8, acc_tile, 0)

    for l in range(8):
      stg_p[0, pl.ds(l * 16, 16)] = jnp.full((16,), g * NBP + SENT + 1,
                                             jnp.int32)

    def wrow(jj, carry):
      def wvreg(l, carry2):
        carry_cum, carry_w = carry2
        j = jj * 8 + l
        v = a1[pl.ds(j * 16, 16)]
        vp = a1[pl.ds(jnp.maximum(j * 16 - 1, 0), 16)]
        k = _node_key(a3, v)
        kp = _node_key(a3, vp)
        neq = (k != kp) | (j == 0)
        mv = plsc.load_gather(a2, [v])
        inc = plsc.cumsum(mv)
        excl2 = inc - mv + carry_cum
        w_in = jnp.where(neq, excl2, -1)
        w_in = jnp.where((iota == 0) & jnp.logical_not(neq), carry_w, w_in)
        wl = plsc.cummax(w_in)
        stg_p[0, pl.ds(l * 16, 16)] = g * NBP + v
        stg_e[0, pl.ds(l * 16, 16)] = wl
        return (carry_cum + jnp.max(inc), jnp.max(wl))
      nv = jnp.minimum(8, NPAD // 16 - jj * 8)
      carry = lax.fori_loop(0, nv, wvreg, carry)
      pltpu.sync_copy(stg_e.at[0], wc_sh.at[stg_p.at[0]])
      return carry
    lax.fori_loop(0, (NPAD // 16 + 7) // 8, wrow,
                  (jnp.int32(0), jnp.int32(0)))

    for l in range(8):
      stg_p[0, pl.ds(l * 16, 16)] = g * NBP + SENT + iota
      stg_e[0, pl.ds(l * 16, 16)] = jnp.full((16,), E_K, jnp.int32)
    pltpu.sync_copy(stg_e.at[0], wc_sh.at[stg_p.at[0]])

  @pl.when(t > 0)
  def _phase3a():
    _zero(a2, NBP)

    def acc_tile(t2, _):
      _add_from_shared(a2, mgrid, (g * 7 + t2 - 1) * NBP, ebuf_r)
      return 0
    lax.fori_loop(1, t, acc_tile, 0)

  plsc.subcore_barrier()

  @pl.when(t > 0)
  def _phase4():
    _add_from_shared(a2, wc_sh, g * NBP, ebuf_r)

    pltpu.async_copy(row_hbm.at[pl.ds(b * EB + c0 * CH, CH)],
                     ebuf_r.at[pl.ds(0, CH)], sem_in)
    pltpu.async_copy(col_hbm.at[pl.ds(b * EB + c0 * CH, CH)],
                     ebuf_c.at[pl.ds(0, CH)], sem_in)

    def drain_row(rr):
      pltpu.make_async_copy(stg_e.at[rr], oute_sh.at[stg_p.at[rr]],
                            sem_out).wait()
      pltpu.make_async_copy(stg_s.at[rr], outs_sh.at[stg_p.at[rr]],
                            sem_out).wait()

    def chunk(ci, _):
      par = lax.rem(ci - c0, 2)
      base_e = b * EB + ci * CH
      pltpu.make_async_copy(row_hbm.at[pl.ds(base_e, CH)],
                            ebuf_r.at[pl.ds(par * CH, CH)], sem_in).wait()
      pltpu.make_async_copy(col_hbm.at[pl.ds(base_e, CH)],
                            ebuf_c.at[pl.ds(par * CH, CH)], sem_in).wait()

      @pl.when(ci + 1 < c0 + nchunks)
      def _pref():
        nbase = b * EB + (ci + 1) * CH
        pltpu.async_copy(row_hbm.at[pl.ds(nbase, CH)],
                         ebuf_r.at[pl.ds((1 - par) * CH, CH)], sem_in)
        pltpu.async_copy(col_hbm.at[pl.ds(nbase, CH)],
                         ebuf_c.at[pl.ds((1 - par) * CH, CH)], sem_in)

      def row(jj, _2):
        rr = lax.rem(jj, NRING)

        def vreg(l, _3):
          j = jj * 8 + l
          r = ebuf_r[pl.ds(par * CH + j * 16, 16)] - b * NB
          c = ebuf_c[pl.ds(par * CH + j * 16, 16)] - b * NB
          rp = plsc.load_gather(a1, [r])
          cp = plsc.load_gather(a1, [c])
          key = jnp.where(rp < 0, cp & 0x7FFFFFFF, SENT)
          cnt, is_last = plsc.scan_count(key)
          base = plsc.load_gather(a2, [key])
          pos = base + cnt - 1
          plsc.store_scatter(a2, [key], base + cnt, mask=is_last)
          outpos = jnp.where(pos < E_K, pos, E_K + (pos & 511))
          stg_p[rr, pl.ds(l * 16, 16)] = g * OUTP + outpos
          stg_e[rr, pl.ds(l * 16, 16)] = base_e + j * 16 + iota
          stg_s[rr, pl.ds(l * 16, 16)] = plsc.load_gather(a3, [key])
          return 0
        lax.fori_loop(0, 8, vreg, 0)
        pltpu.async_copy(stg_e.at[rr], oute_sh.at[stg_p.at[rr]], sem_out)
        pltpu.async_copy(stg_s.at[rr], outs_sh.at[stg_p.at[rr]], sem_out)

        @pl.when(jj >= 4)
        def _ringdrain():
          drain_row(lax.rem(jj - 4, NRING))
        return 0
      lax.fori_loop(0, CH // 128, row, 0)

      def tail_drain(jj, _2):
        drain_row(lax.rem(jj, NRING))
        return 0
      lax.fori_loop(CH // 128 - 4, CH // 128, tail_drain, 0)
      return 0
    lax.fori_loop(c0, c0 + nchunks, chunk, 0)

  plsc.subcore_barrier()

  @pl.when(t == 0)
  def _final():
    def out_chunk(k2, _):
      pltpu.sync_copy(oute_sh.at[pl.ds(g * OUTP + k2 * FCH, FCH)],
                      ebuf_r.at[pl.ds(0, FCH)])
      pltpu.sync_copy(ebuf_r.at[pl.ds(0, FCH)],
                      oute_hbm.at[pl.ds(b * E_K + k2 * FCH, FCH)])
      pltpu.sync_copy(outs_sh.at[pl.ds(g * OUTP + k2 * FCH, FCH)], buf_s)
      pltpu.sync_copy(buf_s, outs_hbm.at[pl.ds(b * E_K + k2 * FCH, FCH)])
      return 0
    lax.fori_loop(0, E_K // FCH, out_chunk, 0)


def kernel(score, edge_index, batch):
  mesh = plsc.VectorSubcoreMesh(core_axis_name="c", subcore_axis_name="s",
                                num_cores=2, num_subcores=16)
  f = pl.kernel(
      _body,
      compiler_params=pltpu.CompilerParams(needs_layout_passes=False),
      out_type=(jax.ShapeDtypeStruct((B * E_K,), jnp.int32),
                jax.ShapeDtypeStruct((B * E_K,), jnp.float32)),
      mesh=mesh,
      scratch_types=[
          pltpu.VMEM((NBP,), jnp.int32),
          pltpu.VMEM((NBP,), jnp.int32),
          pltpu.VMEM((NBP,), jnp.float32),
          pltpu.VMEM((256,), jnp.int32),
          pltpu.VMEM((FCH,), jnp.float32),
          pltpu.VMEM((NRING, 128), jnp.int32),
          pltpu.VMEM((NRING, 128), jnp.int32),
          pltpu.VMEM((NRING, 128), jnp.float32),
          pltpu.VMEM((2 * CH,), jnp.int32),
          pltpu.VMEM((2 * CH,), jnp.int32),
          pltpu.SemaphoreType.DMA,
          pltpu.SemaphoreType.DMA,
          pltpu.VMEM_SHARED((2 * NBP,), jnp.int32),
          pltpu.VMEM_SHARED((2 * NBP,), jnp.int32),
          pltpu.VMEM_SHARED((14 * NBP,), jnp.int32),
          pltpu.VMEM_SHARED((2 * OUTP,), jnp.int32),
          pltpu.VMEM_SHARED((2 * OUTP,), jnp.float32),
      ],
  )
  return f(score, edge_index[0], edge_index[1], batch)


# speedup vs baseline: 405.3506x; 1.1531x over previous
import jax
import jax.numpy as jnp
from jax import lax
from jax.experimental import pallas as pl
from jax.experimental.pallas import tpu as pltpu
from jax.experimental.pallas import tpu_sc as plsc

N = 100000
B = 4
NB = 25000
DEG = 64
EB = NB * DEG
K_B = 2500
E_K = 40000

NPAD = 25008
NBP = 25024
SENT = 25008
CH = 1280
NCH = EB // CH
OUTP = 40960
NRING = 8
FCH = 2000

_CHUNKS = [(i * 2000, 2000) for i in range(12)] + [(24000, 1024)]


def _key_from_score(s):
  bits = lax.bitcast_convert_type(s, jnp.uint32)
  neg = (bits >> jnp.uint32(31)) != jnp.uint32(0)
  u = jnp.where(neg, ~bits, bits | jnp.uint32(0x80000000))
  return ~u


def _node_key(score_ref, v):
  s = plsc.load_gather(score_ref, [v])
  k = _key_from_score(s)
  return jnp.where(v < NB, k, jnp.uint32(0xFFFFFFFF))


def _add_from_shared(dst, src_sh, src_base, buf):
  for off, sz in _CHUNKS:
    pltpu.sync_copy(src_sh.at[pl.ds(src_base + off, sz)],
                    buf.at[pl.ds(0, sz)])

    def addv(k2, _, off=off):
      dst[pl.ds(off + k2 * 16, 16)] = (
          dst[pl.ds(off + k2 * 16, 16)] + buf[pl.ds(k2 * 16, 16)])
      return 0

    lax.fori_loop(0, sz // 16, addv, 0)


def _zero(ref, nwords):
  def z(j, _):
    ref[pl.ds(j * 16, 16)] = jnp.zeros((16,), jnp.int32)
    return 0
  lax.fori_loop(0, nwords // 16, z, 0)


def _body(score_hbm, row_hbm, col_hbm, batch_hbm, oute_hbm, outs_hbm,
          a1, a2, a3, hist, buf_s, stg_p, stg_e, stg_s,
          ebuf_r, ebuf_c, pkey, pcnt, pil, psc, sem_in, sem_out,
          reppk_sh, wc_sh, mgrid, oute_sh, outs_sh):
  del batch_hbm
  cid = lax.axis_index("c")
  sid = lax.axis_index("s")
  g = sid // 8
  t = sid % 8
  b = cid * 2 + g
  iota = lax.iota(jnp.int32, 16)

  pltpu.sync_copy(score_hbm.at[pl.ds(b * NB, NB)], a3.at[pl.ds(0, NB)])

  @pl.when(t == 0)
  def _phase1():
    def init(j, _):
      a1[pl.ds(j * 16, 16)] = j * 16 + iota
      return 0
    lax.fori_loop(0, NPAD // 16, init, 0)

    for p in range(4):
      src, dst = (a1, a2) if p % 2 == 0 else (a2, a1)
      sh = jnp.uint32(8 * p)
      _zero(hist, 256)

      @plsc.parallel_loop(0, NPAD // 16, unroll=4)
      def histo(j, src=src, sh=sh):
        v = src[pl.ds(j * 16, 16)]
        d = ((_node_key(a3, v) >> sh) & jnp.uint32(255)).astype(jnp.int32)
        cnt, is_last = plsc.scan_count(d)
        plsc.addupdate_scatter(hist, [d], cnt, mask=is_last)

      def excl(h, carry):
        vv = hist[pl.ds(h * 16, 16)]
        inc = plsc.cumsum(vv)
        hist[pl.ds(h * 16, 16)] = inc - vv + carry
        return carry + jnp.max(inc)
      lax.fori_loop(0, 16, excl, jnp.int32(0))

      def place(j, _, src=src, dst=dst, sh=sh):
        v = src[pl.ds(j * 16, 16)]
        d = ((_node_key(a3, v) >> sh) & jnp.uint32(255)).astype(jnp.int32)
        cnt, is_last = plsc.scan_count(d)
        base = plsc.load_gather(hist, [d])
        plsc.store_scatter(dst, [base + cnt - 1], v)
        plsc.store_scatter(hist, [d], base + cnt, mask=is_last)
        return 0
      lax.fori_loop(0, NPAD // 16, place, 0)

    def repb(j, carry_rep):
      v = a1[pl.ds(j * 16, 16)]
      vp = a1[pl.ds(jnp.maximum(j * 16 - 1, 0), 16)]
      k = _node_key(a3, v)
      kp = _node_key(a3, vp)
      neq = (k != kp) | (j == 0)
      packed = jnp.where(neq, iota * 32768 + v, -1)
      packed = jnp.where((iota == 0) & jnp.logical_not(neq), carry_rep, packed)
      pm = plsc.cummax(packed)
      rep = pm & 32767
      selbit = jnp.where((j * 16 + iota) < K_B, jnp.int32(-2147483648),
                         jnp.int32(0))
      plsc.store_scatter(a2, [v], rep | selbit)
      return jnp.max(pm) & 32767
    lax.fori_loop(0, NPAD // 16, repb, jnp.int32(0))
    pltpu.sync_copy(a2, reppk_sh.at[pl.ds(g * NBP, NBP)])

  plsc.subcore_barrier()

  c0 = (t - 1) * 178 + jnp.minimum(t - 1, 4)
  nchunks = jnp.where(t <= 4, 179, 178)

  @pl.when(t > 0)
  def _phase2():
    pltpu.sync_copy(reppk_sh.at[pl.ds(g * NBP, NBP)], a1)
    _zero(a2, NBP)

    pltpu.async_copy(row_hbm.at[pl.ds(b * EB + c0 * CH, CH)],
                     ebuf_r.at[pl.ds(0, CH)], sem_in)
    pltpu.async_copy(col_hbm.at[pl.ds(b * EB + c0 * CH, CH)],
                     ebuf_c.at[pl.ds(0, CH)], sem_in)

    def chunk(ci, _):
      par = lax.rem(ci - c0, 2)
      base_e = b * EB + ci * CH
      pltpu.make_async_copy(row_hbm.at[pl.ds(base_e, CH)],
                            ebuf_r.at[pl.ds(par * CH, CH)], sem_in).wait()
      pltpu.make_async_copy(col_hbm.at[pl.ds(base_e, CH)],
                            ebuf_c.at[pl.ds(par * CH, CH)], sem_in).wait()

      @pl.when(ci + 1 < c0 + nchunks)
      def _pref():
        nbase = b * EB + (ci + 1) * CH
        pltpu.async_copy(row_hbm.at[pl.ds(nbase, CH)],
                         ebuf_r.at[pl.ds((1 - par) * CH, CH)], sem_in)
        pltpu.async_copy(col_hbm.at[pl.ds(nbase, CH)],
                         ebuf_c.at[pl.ds((1 - par) * CH, CH)], sem_in)

      @plsc.parallel_loop(0, CH // 16, unroll=4)
      def vreg(j):
        r = ebuf_r[pl.ds(par * CH + j * 16, 16)] - b * NB
        c = ebuf_c[pl.ds(par * CH + j * 16, 16)] - b * NB
        rp = plsc.load_gather(a1, [r])
        cp = plsc.load_gather(a1, [c])
        key = jnp.where(rp < 0, cp & 0x7FFFFFFF, SENT)
        cnt, is_last = plsc.scan_count(key)
        plsc.addupdate_scatter(a2, [key], cnt, mask=is_last)
      return 0
    lax.fori_loop(c0, c0 + nchunks, chunk, 0)
    pltpu.sync_copy(a2, mgrid.at[pl.ds((g * 7 + t - 1) * NBP, NBP)])

  plsc.subcore_barrier()

  @pl.when(t == 0)
  def _phase3_lead():
    pltpu.sync_copy(mgrid.at[pl.ds(g * 7 * NBP, NBP)], a2)

    def acc_tile(t2, _):
      _add_from_shared(a2, mgrid, (g * 7 + t2 - 1) * NBP, ebuf_r)
      return 0
    lax.fori_loop(2, 8, acc_tile, 0)

    for l in range(8):
      stg_p[0, pl.ds(l * 16, 16)] = jnp.full((16,), g * NBP + SENT + 1,
                                             jnp.int32)

    def wrow(jj, carry):
      def wvreg(l, carry2):
        carry_cum, carry_w = carry2
        j = jj * 8 + l
        v = a1[pl.ds(j * 16, 16)]
        vp = a1[pl.ds(jnp.maximum(j * 16 - 1, 0), 16)]
        k = _node_key(a3, v)
        kp = _node_key(a3, vp)
        neq = (k != kp) | (j == 0)
        mv = plsc.load_gather(a2, [v])
        inc = plsc.cumsum(mv)
        excl2 = inc - mv + carry_cum
        w_in = jnp.where(neq, excl2, -1)
        w_in = jnp.where((iota == 0) & jnp.logical_not(neq), carry_w, w_in)
        wl = plsc.cummax(w_in)
        stg_p[0, pl.ds(l * 16, 16)] = g * NBP + v
        stg_e[0, pl.ds(l * 16, 16)] = wl
        return (carry_cum + jnp.max(inc), jnp.max(wl))
      nv = jnp.minimum(8, NPAD // 16 - jj * 8)
      carry = lax.fori_loop(0, nv, wvreg, carry)
      pltpu.sync_copy(stg_e.at[0], wc_sh.at[stg_p.at[0]])
      return carry
    lax.fori_loop(0, (NPAD // 16 + 7) // 8, wrow,
                  (jnp.int32(0), jnp.int32(0)))

    for l in range(8):
      stg_p[0, pl.ds(l * 16, 16)] = g * NBP + SENT + iota
      stg_e[0, pl.ds(l * 16, 16)] = jnp.full((16,), E_K, jnp.int32)
    pltpu.sync_copy(stg_e.at[0], wc_sh.at[stg_p.at[0]])

  @pl.when(t > 0)
  def _phase3a():
    _zero(a2, NBP)

    def acc_tile(t2, _):
      _add_from_shared(a2, mgrid, (g * 7 + t2 - 1) * NBP, ebuf_r)
      return 0
    lax.fori_loop(1, t, acc_tile, 0)

  plsc.subcore_barrier()

  @pl.when(t > 0)
  def _phase4():
    _add_from_shared(a2, wc_sh, g * NBP, ebuf_r)

    pltpu.async_copy(row_hbm.at[pl.ds(b * EB + c0 * CH, CH)],
                     ebuf_r.at[pl.ds(0, CH)], sem_in)
    pltpu.async_copy(col_hbm.at[pl.ds(b * EB + c0 * CH, CH)],
                     ebuf_c.at[pl.ds(0, CH)], sem_in)

    def drain_row(rr):
      pltpu.make_async_copy(stg_e.at[rr], oute_sh.at[stg_p.at[rr]],
                            sem_out).wait()
      pltpu.make_async_copy(stg_s.at[rr], outs_sh.at[stg_p.at[rr]],
                            sem_out).wait()

    def chunk(ci, _):
      par = lax.rem(ci - c0, 2)
      base_e = b * EB + ci * CH
      pltpu.make_async_copy(row_hbm.at[pl.ds(base_e, CH)],
                            ebuf_r.at[pl.ds(par * CH, CH)], sem_in).wait()
      pltpu.make_async_copy(col_hbm.at[pl.ds(base_e, CH)],
                            ebuf_c.at[pl.ds(par * CH, CH)], sem_in).wait()

      @pl.when(ci + 1 < c0 + nchunks)
      def _pref():
        nbase = b * EB + (ci + 1) * CH
        pltpu.async_copy(row_hbm.at[pl.ds(nbase, CH)],
                         ebuf_r.at[pl.ds((1 - par) * CH, CH)], sem_in)
        pltpu.async_copy(col_hbm.at[pl.ds(nbase, CH)],
                         ebuf_c.at[pl.ds((1 - par) * CH, CH)], sem_in)

      @plsc.parallel_loop(0, CH // 16, unroll=4)
      def prep(j):
        r = ebuf_r[pl.ds(par * CH + j * 16, 16)] - b * NB
        c = ebuf_c[pl.ds(par * CH + j * 16, 16)] - b * NB
        rp = plsc.load_gather(a1, [r])
        cp = plsc.load_gather(a1, [c])
        key = jnp.where(rp < 0, cp & 0x7FFFFFFF, SENT)
        cnt, is_last = plsc.scan_count(key)
        pkey[pl.ds(j * 16, 16)] = key
        pcnt[pl.ds(j * 16, 16)] = cnt
        pil[pl.ds(j * 16, 16)] = jnp.where(is_last, 1, 0)
        psc[pl.ds(j * 16, 16)] = plsc.load_gather(a3, [key])

      def row(jj, _2):
        rr = lax.rem(jj, NRING)

        def vreg(l, _3):
          j = jj * 8 + l
          key = pkey[pl.ds(j * 16, 16)]
          cnt = pcnt[pl.ds(j * 16, 16)]
          il = pil[pl.ds(j * 16, 16)] != 0
          base = plsc.load_gather(a2, [key])
          pos = base + cnt - 1
          plsc.store_scatter(a2, [key], base + cnt, mask=il)
          outpos = jnp.where(pos < E_K, pos, E_K + (pos & 511))
          stg_p[rr, pl.ds(l * 16, 16)] = g * OUTP + outpos
          stg_e[rr, pl.ds(l * 16, 16)] = base_e + j * 16 + iota
          stg_s[rr, pl.ds(l * 16, 16)] = psc[pl.ds(j * 16, 16)]
          return 0
        lax.fori_loop(0, 8, vreg, 0)
        pltpu.async_copy(stg_e.at[rr], oute_sh.at[stg_p.at[rr]], sem_out)
        pltpu.async_copy(stg_s.at[rr], outs_sh.at[stg_p.at[rr]], sem_out)

        @pl.when(jj >= 4)
        def _ringdrain():
          drain_row(lax.rem(jj - 4, NRING))
        return 0
      lax.fori_loop(0, CH // 128, row, 0)

      def tail_drain(jj, _2):
        drain_row(lax.rem(jj, NRING))
        return 0
      lax.fori_loop(CH // 128 - 4, CH // 128, tail_drain, 0)
      return 0
    lax.fori_loop(c0, c0 + nchunks, chunk, 0)

  plsc.subcore_barrier()

  @pl.when(t == 0)
  def _final():
    def out_chunk(k2, _):
      pltpu.sync_copy(oute_sh.at[pl.ds(g * OUTP + k2 * FCH, FCH)],
                      ebuf_r.at[pl.ds(0, FCH)])
      pltpu.sync_copy(ebuf_r.at[pl.ds(0, FCH)],
                      oute_hbm.at[pl.ds(b * E_K + k2 * FCH, FCH)])
      pltpu.sync_copy(outs_sh.at[pl.ds(g * OUTP + k2 * FCH, FCH)], buf_s)
      pltpu.sync_copy(buf_s, outs_hbm.at[pl.ds(b * E_K + k2 * FCH, FCH)])
      return 0
    lax.fori_loop(0, E_K // FCH, out_chunk, 0)


def kernel(score, edge_index, batch):
  mesh = plsc.VectorSubcoreMesh(core_axis_name="c", subcore_axis_name="s",
                                num_cores=2, num_subcores=16)
  f = pl.kernel(
      _body,
      compiler_params=pltpu.CompilerParams(needs_layout_passes=False),
      out_type=(jax.ShapeDtypeStruct((B * E_K,), jnp.int32),
                jax.ShapeDtypeStruct((B * E_K,), jnp.float32)),
      mesh=mesh,
      scratch_types=[
          pltpu.VMEM((NBP,), jnp.int32),
          pltpu.VMEM((NBP,), jnp.int32),
          pltpu.VMEM((NBP,), jnp.float32),
          pltpu.VMEM((256,), jnp.int32),
          pltpu.VMEM((FCH,), jnp.float32),
          pltpu.VMEM((NRING, 128), jnp.int32),
          pltpu.VMEM((NRING, 128), jnp.int32),
          pltpu.VMEM((NRING, 128), jnp.float32),
          pltpu.VMEM((2 * CH,), jnp.int32),
          pltpu.VMEM((2 * CH,), jnp.int32),
          pltpu.VMEM((CH,), jnp.int32),
          pltpu.VMEM((CH,), jnp.int32),
          pltpu.VMEM((CH,), jnp.int32),
          pltpu.VMEM((CH,), jnp.float32),
          pltpu.SemaphoreType.DMA,
          pltpu.SemaphoreType.DMA,
          pltpu.VMEM_SHARED((2 * NBP,), jnp.int32),
          pltpu.VMEM_SHARED((2 * NBP,), jnp.int32),
          pltpu.VMEM_SHARED((14 * NBP,), jnp.int32),
          pltpu.VMEM_SHARED((2 * OUTP,), jnp.int32),
          pltpu.VMEM_SHARED((2 * OUTP,), jnp.float32),
      ],
  )
  return f(score, edge_index[0], edge_index[1], batch)


# speedup vs baseline: 431.8897x; 1.0655x over previous
import jax
import jax.numpy as jnp
from jax import lax
from jax.experimental import pallas as pl
from jax.experimental.pallas import tpu as pltpu
from jax.experimental.pallas import tpu_sc as plsc

N = 100000
B = 4
NB = 25000
DEG = 64
EB = NB * DEG
K_B = 2500
E_K = 40000

NPAD = 25008
NBP = 25024
SENT = 25008
CH = 1280
NCH = EB // CH
OUTP = 40960
NRING = 8
FCH = 2000

_CHUNKS = [(i * 2000, 2000) for i in range(12)] + [(24000, 1024)]


def _key_from_score(s):
  bits = lax.bitcast_convert_type(s, jnp.uint32)
  neg = (bits >> jnp.uint32(31)) != jnp.uint32(0)
  u = jnp.where(neg, ~bits, bits | jnp.uint32(0x80000000))
  return ~u


def _node_key(score_ref, v):
  s = plsc.load_gather(score_ref, [v])
  k = _key_from_score(s)
  return jnp.where(v < NB, k, jnp.uint32(0xFFFFFFFF))


def _add_from_shared(dst, src_sh, src_base, buf):
  for off, sz in _CHUNKS:
    pltpu.sync_copy(src_sh.at[pl.ds(src_base + off, sz)],
                    buf.at[pl.ds(0, sz)])

    def addv(k2, _, off=off):
      dst[pl.ds(off + k2 * 16, 16)] = (
          dst[pl.ds(off + k2 * 16, 16)] + buf[pl.ds(k2 * 16, 16)])
      return 0

    lax.fori_loop(0, sz // 16, addv, 0)


def _zero(ref, nwords):
  def z(j, _):
    ref[pl.ds(j * 16, 16)] = jnp.zeros((16,), jnp.int32)
    return 0
  lax.fori_loop(0, nwords // 16, z, 0)


def _body(score_hbm, row_hbm, col_hbm, batch_hbm, oute_hbm, outs_hbm,
          a1, a2, a3, hist, buf_s, stg_p, stg_e, stg_s,
          ebuf_r, ebuf_c, pkey, pcnt, pil, psc, pve, sem_in, sem_out,
          reppk_sh, wc_sh, mgrid, oute_sh, outs_sh):
  del batch_hbm
  cid = lax.axis_index("c")
  sid = lax.axis_index("s")
  g = sid // 8
  t = sid % 8
  b = cid * 2 + g
  iota = lax.iota(jnp.int32, 16)

  pltpu.sync_copy(score_hbm.at[pl.ds(b * NB, NB)], a3.at[pl.ds(0, NB)])

  @pl.when(t == 0)
  def _phase1():
    def init(j, _):
      a1[pl.ds(j * 16, 16)] = j * 16 + iota
      return 0
    lax.fori_loop(0, NPAD // 16, init, 0)

    for p in range(4):
      src, dst = (a1, a2) if p % 2 == 0 else (a2, a1)
      sh = jnp.uint32(8 * p)
      _zero(hist, 256)

      @plsc.parallel_loop(0, NPAD // 16, unroll=4)
      def histo(j, src=src, sh=sh):
        v = src[pl.ds(j * 16, 16)]
        d = ((_node_key(a3, v) >> sh) & jnp.uint32(255)).astype(jnp.int32)
        cnt, is_last = plsc.scan_count(d)
        plsc.addupdate_scatter(hist, [d], cnt, mask=is_last)

      def excl(h, carry):
        vv = hist[pl.ds(h * 16, 16)]
        inc = plsc.cumsum(vv)
        hist[pl.ds(h * 16, 16)] = inc - vv + carry
        return carry + jnp.max(inc)
      lax.fori_loop(0, 16, excl, jnp.int32(0))

      def qchunk(q, _, src=src, dst=dst, sh=sh):
        nv = jnp.minimum(80, NPAD // 16 - q * 80)

        @plsc.parallel_loop(0, nv, unroll=4)
        def prep2(jl):
          v = src[pl.ds((q * 80 + jl) * 16, 16)]
          d = ((_node_key(a3, v) >> sh) & jnp.uint32(255)).astype(jnp.int32)
          cnt, is_last = plsc.scan_count(d)
          pkey[pl.ds(jl * 16, 16)] = d
          pcnt[pl.ds(jl * 16, 16)] = cnt
          pil[pl.ds(jl * 16, 16)] = jnp.where(is_last, 1, 0)
          pve[pl.ds(jl * 16, 16)] = v

        def place(jl, _2):
          d = pkey[pl.ds(jl * 16, 16)]
          cnt = pcnt[pl.ds(jl * 16, 16)]
          il = pil[pl.ds(jl * 16, 16)] != 0
          v = pve[pl.ds(jl * 16, 16)]
          base = plsc.load_gather(hist, [d])
          plsc.store_scatter(dst, [base + cnt - 1], v)
          plsc.store_scatter(hist, [d], base + cnt, mask=il)
          return 0
        lax.fori_loop(0, nv, place, 0)
        return 0
      lax.fori_loop(0, (NPAD // 16 + 79) // 80, qchunk, 0)

    def repb(j, carry_rep):
      v = a1[pl.ds(j * 16, 16)]
      vp = a1[pl.ds(jnp.maximum(j * 16 - 1, 0), 16)]
      k = _node_key(a3, v)
      kp = _node_key(a3, vp)
      neq = (k != kp) | (j == 0)
      packed = jnp.where(neq, iota * 32768 + v, -1)
      packed = jnp.where((iota == 0) & jnp.logical_not(neq), carry_rep, packed)
      pm = plsc.cummax(packed)
      rep = pm & 32767
      selbit = jnp.where((j * 16 + iota) < K_B, jnp.int32(-2147483648),
                         jnp.int32(0))
      plsc.store_scatter(a2, [v], rep | selbit)
      return jnp.max(pm) & 32767
    lax.fori_loop(0, NPAD // 16, repb, jnp.int32(0))
    pltpu.sync_copy(a2, reppk_sh.at[pl.ds(g * NBP, NBP)])

  plsc.subcore_barrier()

  c0 = (t - 1) * 178 + jnp.minimum(t - 1, 4)
  nchunks = jnp.where(t <= 4, 179, 178)

  @pl.when(t > 0)
  def _phase2():
    pltpu.sync_copy(reppk_sh.at[pl.ds(g * NBP, NBP)], a1)
    _zero(a2, NBP)

    pltpu.async_copy(row_hbm.at[pl.ds(b * EB + c0 * CH, CH)],
                     ebuf_r.at[pl.ds(0, CH)], sem_in)
    pltpu.async_copy(col_hbm.at[pl.ds(b * EB + c0 * CH, CH)],
                     ebuf_c.at[pl.ds(0, CH)], sem_in)

    def chunk(ci, _):
      par = lax.rem(ci - c0, 2)
      base_e = b * EB + ci * CH
      pltpu.make_async_copy(row_hbm.at[pl.ds(base_e, CH)],
                            ebuf_r.at[pl.ds(par * CH, CH)], sem_in).wait()
      pltpu.make_async_copy(col_hbm.at[pl.ds(base_e, CH)],
                            ebuf_c.at[pl.ds(par * CH, CH)], sem_in).wait()

      @pl.when(ci + 1 < c0 + nchunks)
      def _pref():
        nbase = b * EB + (ci + 1) * CH
        pltpu.async_copy(row_hbm.at[pl.ds(nbase, CH)],
                         ebuf_r.at[pl.ds((1 - par) * CH, CH)], sem_in)
        pltpu.async_copy(col_hbm.at[pl.ds(nbase, CH)],
                         ebuf_c.at[pl.ds((1 - par) * CH, CH)], sem_in)

      @plsc.parallel_loop(0, CH // 16, unroll=4)
      def vreg(j):
        r = ebuf_r[pl.ds(par * CH + j * 16, 16)] - b * NB
        c = ebuf_c[pl.ds(par * CH + j * 16, 16)] - b * NB
        rp = plsc.load_gather(a1, [r])
        cp = plsc.load_gather(a1, [c])
        key = jnp.where(rp < 0, cp & 0x7FFFFFFF, SENT)
        cnt, is_last = plsc.scan_count(key)
        plsc.addupdate_scatter(a2, [key], cnt, mask=is_last)
      return 0
    lax.fori_loop(c0, c0 + nchunks, chunk, 0)
    pltpu.sync_copy(a2, mgrid.at[pl.ds((g * 7 + t - 1) * NBP, NBP)])

  plsc.subcore_barrier()

  @pl.when(t == 0)
  def _phase3_lead():
    pltpu.sync_copy(mgrid.at[pl.ds(g * 7 * NBP, NBP)], a2)

    def acc_tile(t2, _):
      _add_from_shared(a2, mgrid, (g * 7 + t2 - 1) * NBP, ebuf_r)
      return 0
    lax.fori_loop(2, 8, acc_tile, 0)

    for l in range(8):
      stg_p[0, pl.ds(l * 16, 16)] = jnp.full((16,), g * NBP + SENT + 1,
                                             jnp.int32)

    def wrow(jj, carry):
      def wvreg(l, carry2):
        carry_cum, carry_w = carry2
        j = jj * 8 + l
        v = a1[pl.ds(j * 16, 16)]
        vp = a1[pl.ds(jnp.maximum(j * 16 - 1, 0), 16)]
        k = _node_key(a3, v)
        kp = _node_key(a3, vp)
        neq = (k != kp) | (j == 0)
        mv = plsc.load_gather(a2, [v])
        inc = plsc.cumsum(mv)
        excl2 = inc - mv + carry_cum
        w_in = jnp.where(neq, excl2, -1)
        w_in = jnp.where((iota == 0) & jnp.logical_not(neq), carry_w, w_in)
        wl = plsc.cummax(w_in)
        stg_p[0, pl.ds(l * 16, 16)] = g * NBP + v
        stg_e[0, pl.ds(l * 16, 16)] = wl
        return (carry_cum + jnp.max(inc), jnp.max(wl))
      nv = jnp.minimum(8, NPAD // 16 - jj * 8)
      carry = lax.fori_loop(0, nv, wvreg, carry)
      pltpu.sync_copy(stg_e.at[0], wc_sh.at[stg_p.at[0]])
      return carry
    lax.fori_loop(0, (NPAD // 16 + 7) // 8, wrow,
                  (jnp.int32(0), jnp.int32(0)))

    for l in range(8):
      stg_p[0, pl.ds(l * 16, 16)] = g * NBP + SENT + iota
      stg_e[0, pl.ds(l * 16, 16)] = jnp.full((16,), E_K, jnp.int32)
    pltpu.sync_copy(stg_e.at[0], wc_sh.at[stg_p.at[0]])

  @pl.when(t > 0)
  def _phase3a():
    _zero(a2, NBP)

    def acc_tile(t2, _):
      _add_from_shared(a2, mgrid, (g * 7 + t2 - 1) * NBP, ebuf_r)
      return 0
    lax.fori_loop(1, t, acc_tile, 0)

  plsc.subcore_barrier()

  @pl.when(t > 0)
  def _phase4():
    _add_from_shared(a2, wc_sh, g * NBP, ebuf_r)

    pltpu.async_copy(row_hbm.at[pl.ds(b * EB + c0 * CH, CH)],
                     ebuf_r.at[pl.ds(0, CH)], sem_in)
    pltpu.async_copy(col_hbm.at[pl.ds(b * EB + c0 * CH, CH)],
                     ebuf_c.at[pl.ds(0, CH)], sem_in)

    def drain_row(rr):
      pltpu.make_async_copy(stg_e.at[rr], oute_sh.at[stg_p.at[rr]],
                            sem_out).wait()
      pltpu.make_async_copy(stg_s.at[rr], outs_sh.at[stg_p.at[rr]],
                            sem_out).wait()

    def chunk(ci, _):
      par = lax.rem(ci - c0, 2)
      base_e = b * EB + ci * CH
      pltpu.make_async_copy(row_hbm.at[pl.ds(base_e, CH)],
                            ebuf_r.at[pl.ds(par * CH, CH)], sem_in).wait()
      pltpu.make_async_copy(col_hbm.at[pl.ds(base_e, CH)],
                            ebuf_c.at[pl.ds(par * CH, CH)], sem_in).wait()

      @pl.when(ci + 1 < c0 + nchunks)
      def _pref():
        nbase = b * EB + (ci + 1) * CH
        pltpu.async_copy(row_hbm.at[pl.ds(nbase, CH)],
                         ebuf_r.at[pl.ds((1 - par) * CH, CH)], sem_in)
        pltpu.async_copy(col_hbm.at[pl.ds(nbase, CH)],
                         ebuf_c.at[pl.ds((1 - par) * CH, CH)], sem_in)

      @plsc.parallel_loop(0, CH // 16, unroll=4)
      def prep(j):
        r = ebuf_r[pl.ds(par * CH + j * 16, 16)] - b * NB
        c = ebuf_c[pl.ds(par * CH + j * 16, 16)] - b * NB
        rp = plsc.load_gather(a1, [r])
        cp = plsc.load_gather(a1, [c])
        key = jnp.where(rp < 0, cp & 0x7FFFFFFF, SENT)
        cnt, is_last = plsc.scan_count(key)
        pkey[pl.ds(j * 16, 16)] = key
        pcnt[pl.ds(j * 16, 16)] = cnt
        pil[pl.ds(j * 16, 16)] = jnp.where(is_last, 1, 0)
        psc[pl.ds(j * 16, 16)] = plsc.load_gather(a3, [key])

      def row(jj, _2):
        rr = lax.rem(jj, NRING)

        def vreg(l, _3):
          j = jj * 8 + l
          key = pkey[pl.ds(j * 16, 16)]
          cnt = pcnt[pl.ds(j * 16, 16)]
          il = pil[pl.ds(j * 16, 16)] != 0
          base = plsc.load_gather(a2, [key])
          pos = base + cnt - 1
          plsc.store_scatter(a2, [key], base + cnt, mask=il)
          outpos = jnp.where(pos < E_K, pos, E_K + (pos & 511))
          stg_p[rr, pl.ds(l * 16, 16)] = g * OUTP + outpos
          stg_e[rr, pl.ds(l * 16, 16)] = base_e + j * 16 + iota
          stg_s[rr, pl.ds(l * 16, 16)] = psc[pl.ds(j * 16, 16)]
          return 0
        lax.fori_loop(0, 8, vreg, 0)
        pltpu.async_copy(stg_e.at[rr], oute_sh.at[stg_p.at[rr]], sem_out)
        pltpu.async_copy(stg_s.at[rr], outs_sh.at[stg_p.at[rr]], sem_out)

        @pl.when(jj >= 4)
        def _ringdrain():
          drain_row(lax.rem(jj - 4, NRING))
        return 0
      lax.fori_loop(0, CH // 128, row, 0)

      def tail_drain(jj, _2):
        drain_row(lax.rem(jj, NRING))
        return 0
      lax.fori_loop(CH // 128 - 4, CH // 128, tail_drain, 0)
      return 0
    lax.fori_loop(c0, c0 + nchunks, chunk, 0)

  plsc.subcore_barrier()

  @pl.when(t == 0)
  def _final():
    def out_chunk(k2, _):
      pltpu.sync_copy(oute_sh.at[pl.ds(g * OUTP + k2 * FCH, FCH)],
                      ebuf_r.at[pl.ds(0, FCH)])
      pltpu.sync_copy(ebuf_r.at[pl.ds(0, FCH)],
                      oute_hbm.at[pl.ds(b * E_K + k2 * FCH, FCH)])
      pltpu.sync_copy(outs_sh.at[pl.ds(g * OUTP + k2 * FCH, FCH)], buf_s)
      pltpu.sync_copy(buf_s, outs_hbm.at[pl.ds(b * E_K + k2 * FCH, FCH)])
      return 0
    lax.fori_loop(0, E_K // FCH, out_chunk, 0)


def kernel(score, edge_index, batch):
  mesh = plsc.VectorSubcoreMesh(core_axis_name="c", subcore_axis_name="s",
                                num_cores=2, num_subcores=16)
  f = pl.kernel(
      _body,
      compiler_params=pltpu.CompilerParams(needs_layout_passes=False),
      out_type=(jax.ShapeDtypeStruct((B * E_K,), jnp.int32),
                jax.ShapeDtypeStruct((B * E_K,), jnp.float32)),
      mesh=mesh,
      scratch_types=[
          pltpu.VMEM((NBP,), jnp.int32),
          pltpu.VMEM((NBP,), jnp.int32),
          pltpu.VMEM((NBP,), jnp.float32),
          pltpu.VMEM((256,), jnp.int32),
          pltpu.VMEM((FCH,), jnp.float32),
          pltpu.VMEM((NRING, 128), jnp.int32),
          pltpu.VMEM((NRING, 128), jnp.int32),
          pltpu.VMEM((NRING, 128), jnp.float32),
          pltpu.VMEM((2 * CH,), jnp.int32),
          pltpu.VMEM((2 * CH,), jnp.int32),
          pltpu.VMEM((CH,), jnp.int32),
          pltpu.VMEM((CH,), jnp.int32),
          pltpu.VMEM((CH,), jnp.int32),
          pltpu.VMEM((CH,), jnp.float32),
          pltpu.VMEM((CH,), jnp.int32),
          pltpu.SemaphoreType.DMA,
          pltpu.SemaphoreType.DMA,
          pltpu.VMEM_SHARED((2 * NBP,), jnp.int32),
          pltpu.VMEM_SHARED((2 * NBP,), jnp.int32),
          pltpu.VMEM_SHARED((14 * NBP,), jnp.int32),
          pltpu.VMEM_SHARED((2 * OUTP,), jnp.int32),
          pltpu.VMEM_SHARED((2 * OUTP,), jnp.float32),
      ],
  )
  return f(score, edge_index[0], edge_index[1], batch)


# speedup vs baseline: 443.7477x; 1.0275x over previous
import jax
import jax.numpy as jnp
from jax import lax
from jax.experimental import pallas as pl
from jax.experimental.pallas import tpu as pltpu
from jax.experimental.pallas import tpu_sc as plsc

N = 100000
B = 4
NB = 25000
DEG = 64
EB = NB * DEG
K_B = 2500
E_K = 40000

NPAD = 25008
NBP = 25024
SENT = 25008
CH = 1280
NCH = EB // CH
OUTP = 40960
NRING = 8
FCH = 2000

_CHUNKS = [(i * 2000, 2000) for i in range(12)] + [(24000, 1024)]


def _key_from_score(s):
  bits = lax.bitcast_convert_type(s, jnp.uint32)
  neg = (bits >> jnp.uint32(31)) != jnp.uint32(0)
  u = jnp.where(neg, ~bits, bits | jnp.uint32(0x80000000))
  return ~u


def _node_key(score_ref, v):
  s = plsc.load_gather(score_ref, [v])
  k = _key_from_score(s)
  return jnp.where(v < NB, k, jnp.uint32(0xFFFFFFFF))


def _add_from_shared(dst, src_sh, src_base, buf):
  for off, sz in _CHUNKS:
    pltpu.sync_copy(src_sh.at[pl.ds(src_base + off, sz)],
                    buf.at[pl.ds(0, sz)])

    def addv(k2, _, off=off):
      dst[pl.ds(off + k2 * 16, 16)] = (
          dst[pl.ds(off + k2 * 16, 16)] + buf[pl.ds(k2 * 16, 16)])
      return 0

    lax.fori_loop(0, sz // 16, addv, 0)


def _zero(ref, nwords):
  def z(j, _):
    ref[pl.ds(j * 16, 16)] = jnp.zeros((16,), jnp.int32)
    return 0
  lax.fori_loop(0, nwords // 16, z, 0)


def _body(score_hbm, row_hbm, col_hbm, batch_hbm, oute_hbm, outs_hbm,
          a1, a2, a3, hist, buf_s, stg_p, stg_e, stg_s,
          ebuf_r, ebuf_c, pkey, pcnt, pil, psc, sem_in, sem_out,
          reppk_sh, wc_sh, mgrid, oute_sh, outs_sh):
  del batch_hbm
  cid = lax.axis_index("c")
  sid = lax.axis_index("s")
  g = sid // 8
  t = sid % 8
  b = cid * 2 + g
  iota = lax.iota(jnp.int32, 16)

  pltpu.sync_copy(score_hbm.at[pl.ds(b * NB, NB)], a3.at[pl.ds(0, NB)])

  @pl.when(t == 0)
  def _phase1():
    def init(j, _):
      a2[pl.ds(j * 16, 16)] = j * 16 + iota
      return 0
    lax.fori_loop(0, NPAD // 16, init, 0)

    for p in range(3):
      src, dst = (a2, a1) if p % 2 == 0 else (a1, a2)
      sh = jnp.uint32(11 * p)
      _zero(hist, 2048)

      @plsc.parallel_loop(0, NPAD // 16, unroll=4)
      def histo(j, src=src, sh=sh):
        v = src[pl.ds(j * 16, 16)]
        d = ((_node_key(a3, v) >> sh) & jnp.uint32(2047)).astype(jnp.int32)
        cnt, is_last = plsc.scan_count(d)
        plsc.addupdate_scatter(hist, [d], cnt, mask=is_last)

      def excl(h, carry):
        vv = hist[pl.ds(h * 16, 16)]
        inc = plsc.cumsum(vv)
        hist[pl.ds(h * 16, 16)] = inc - vv + carry
        return carry + jnp.max(inc)
      lax.fori_loop(0, 128, excl, jnp.int32(0))

      def qchunk(q, _, src=src, dst=dst, sh=sh):
        nv = jnp.minimum(80, NPAD // 16 - q * 80)

        @plsc.parallel_loop(0, nv, unroll=4)
        def prep2(jl):
          v = src[pl.ds((q * 80 + jl) * 16, 16)]
          d = ((_node_key(a3, v) >> sh) & jnp.uint32(2047)).astype(jnp.int32)
          cnt, is_last = plsc.scan_count(d)
          pkey[pl.ds(jl * 16, 16)] = d
          pcnt[pl.ds(jl * 16, 16)] = cnt
          pil[pl.ds(jl * 16, 16)] = jnp.where(is_last, 1, 0)
          psc[pl.ds(jl * 16, 16)] = plsc.bitcast(v, jnp.float32)

        def place(jl, _2):
          d = pkey[pl.ds(jl * 16, 16)]
          cnt = pcnt[pl.ds(jl * 16, 16)]
          il = pil[pl.ds(jl * 16, 16)] != 0
          v = plsc.bitcast(psc[pl.ds(jl * 16, 16)], jnp.int32)
          base = plsc.load_gather(hist, [d])
          plsc.store_scatter(dst, [base + cnt - 1], v)
          plsc.store_scatter(hist, [d], base + cnt, mask=il)
          return 0
        lax.fori_loop(0, nv, place, 0)
        return 0
      lax.fori_loop(0, (NPAD // 16 + 79) // 80, qchunk, 0)

    def repb(j, carry_rep):
      v = a1[pl.ds(j * 16, 16)]
      vp = a1[pl.ds(jnp.maximum(j * 16 - 1, 0), 16)]
      k = _node_key(a3, v)
      kp = _node_key(a3, vp)
      neq = (k != kp) | (j == 0)
      packed = jnp.where(neq, iota * 32768 + v, -1)
      packed = jnp.where((iota == 0) & jnp.logical_not(neq), carry_rep, packed)
      pm = plsc.cummax(packed)
      rep = pm & 32767
      selbit = jnp.where((j * 16 + iota) < K_B, jnp.int32(-2147483648),
                         jnp.int32(0))
      plsc.store_scatter(a2, [v], rep | selbit)
      return jnp.max(pm) & 32767
    lax.fori_loop(0, NPAD // 16, repb, jnp.int32(0))
    pltpu.sync_copy(a2, reppk_sh.at[pl.ds(g * NBP, NBP)])

  plsc.subcore_barrier()

  c0 = (t - 1) * 178 + jnp.minimum(t - 1, 4)
  nchunks = jnp.where(t <= 4, 179, 178)

  @pl.when(t > 0)
  def _phase2():
    pltpu.sync_copy(reppk_sh.at[pl.ds(g * NBP, NBP)], a1)
    _zero(a2, NBP)

    pltpu.async_copy(row_hbm.at[pl.ds(b * EB + c0 * CH, CH)],
                     ebuf_r.at[pl.ds(0, CH)], sem_in)
    pltpu.async_copy(col_hbm.at[pl.ds(b * EB + c0 * CH, CH)],
                     ebuf_c.at[pl.ds(0, CH)], sem_in)

    def chunk(ci, _):
      par = lax.rem(ci - c0, 2)
      base_e = b * EB + ci * CH
      pltpu.make_async_copy(row_hbm.at[pl.ds(base_e, CH)],
                            ebuf_r.at[pl.ds(par * CH, CH)], sem_in).wait()
      pltpu.make_async_copy(col_hbm.at[pl.ds(base_e, CH)],
                            ebuf_c.at[pl.ds(par * CH, CH)], sem_in).wait()

      @pl.when(ci + 1 < c0 + nchunks)
      def _pref():
        nbase = b * EB + (ci + 1) * CH
        pltpu.async_copy(row_hbm.at[pl.ds(nbase, CH)],
                         ebuf_r.at[pl.ds((1 - par) * CH, CH)], sem_in)
        pltpu.async_copy(col_hbm.at[pl.ds(nbase, CH)],
                         ebuf_c.at[pl.ds((1 - par) * CH, CH)], sem_in)

      @plsc.parallel_loop(0, CH // 16, unroll=8)
      def vreg(j):
        r = ebuf_r[pl.ds(par * CH + j * 16, 16)] - b * NB
        c = ebuf_c[pl.ds(par * CH + j * 16, 16)] - b * NB
        rp = plsc.load_gather(a1, [r])
        cp = plsc.load_gather(a1, [c])
        key = jnp.where(rp < 0, cp & 0x7FFFFFFF, SENT)
        cnt, is_last = plsc.scan_count(key)
        plsc.addupdate_scatter(a2, [key], cnt, mask=is_last)
      return 0
    lax.fori_loop(c0, c0 + nchunks, chunk, 0)
    pltpu.sync_copy(a2, mgrid.at[pl.ds((g * 7 + t - 1) * NBP, NBP)])

  plsc.subcore_barrier()

  @pl.when(t == 0)
  def _phase3_lead():
    pltpu.sync_copy(mgrid.at[pl.ds(g * 7 * NBP, NBP)], a2)

    def acc_tile(t2, _):
      _add_from_shared(a2, mgrid, (g * 7 + t2 - 1) * NBP, ebuf_r)
      return 0
    lax.fori_loop(2, 8, acc_tile, 0)

    for l in range(8):
      stg_p[0, pl.ds(l * 16, 16)] = jnp.full((16,), g * NBP + SENT + 1,
                                             jnp.int32)

    def wrow(jj, carry):
      def wvreg(l, carry2):
        carry_cum, carry_w = carry2
        j = jj * 8 + l
        v = a1[pl.ds(j * 16, 16)]
        vp = a1[pl.ds(jnp.maximum(j * 16 - 1, 0), 16)]
        k = _node_key(a3, v)
        kp = _node_key(a3, vp)
        neq = (k != kp) | (j == 0)
        mv = plsc.load_gather(a2, [v])
        inc = plsc.cumsum(mv)
        excl2 = inc - mv + carry_cum
        w_in = jnp.where(neq, excl2, -1)
        w_in = jnp.where((iota == 0) & jnp.logical_not(neq), carry_w, w_in)
        wl = plsc.cummax(w_in)
        stg_p[0, pl.ds(l * 16, 16)] = g * NBP + v
        stg_e[0, pl.ds(l * 16, 16)] = wl
        return (carry_cum + jnp.max(inc), jnp.max(wl))
      nv = jnp.minimum(8, NPAD // 16 - jj * 8)
      carry = lax.fori_loop(0, nv, wvreg, carry)
      pltpu.sync_copy(stg_e.at[0], wc_sh.at[stg_p.at[0]])
      return carry
    lax.fori_loop(0, (NPAD // 16 + 7) // 8, wrow,
                  (jnp.int32(0), jnp.int32(0)))

    for l in range(8):
      stg_p[0, pl.ds(l * 16, 16)] = g * NBP + SENT + iota
      stg_e[0, pl.ds(l * 16, 16)] = jnp.full((16,), E_K, jnp.int32)
    pltpu.sync_copy(stg_e.at[0], wc_sh.at[stg_p.at[0]])

  @pl.when(t > 0)
  def _phase3a():
    _zero(a2, NBP)

    def acc_tile(t2, _):
      _add_from_shared(a2, mgrid, (g * 7 + t2 - 1) * NBP, ebuf_r)
      return 0
    lax.fori_loop(1, t, acc_tile, 0)

  plsc.subcore_barrier()

  @pl.when(t > 0)
  def _phase4():
    _add_from_shared(a2, wc_sh, g * NBP, ebuf_r)

    pltpu.async_copy(row_hbm.at[pl.ds(b * EB + c0 * CH, CH)],
                     ebuf_r.at[pl.ds(0, CH)], sem_in)
    pltpu.async_copy(col_hbm.at[pl.ds(b * EB + c0 * CH, CH)],
                     ebuf_c.at[pl.ds(0, CH)], sem_in)

    def drain_row(rr):
      pltpu.make_async_copy(stg_e.at[rr], oute_sh.at[stg_p.at[rr]],
                            sem_out).wait()
      pltpu.make_async_copy(stg_s.at[rr], outs_sh.at[stg_p.at[rr]],
                            sem_out).wait()

    def chunk(ci, _):
      par = lax.rem(ci - c0, 2)
      base_e = b * EB + ci * CH
      pltpu.make_async_copy(row_hbm.at[pl.ds(base_e, CH)],
                            ebuf_r.at[pl.ds(par * CH, CH)], sem_in).wait()
      pltpu.make_async_copy(col_hbm.at[pl.ds(base_e, CH)],
                            ebuf_c.at[pl.ds(par * CH, CH)], sem_in).wait()

      @pl.when(ci + 1 < c0 + nchunks)
      def _pref():
        nbase = b * EB + (ci + 1) * CH
        pltpu.async_copy(row_hbm.at[pl.ds(nbase, CH)],
                         ebuf_r.at[pl.ds((1 - par) * CH, CH)], sem_in)
        pltpu.async_copy(col_hbm.at[pl.ds(nbase, CH)],
                         ebuf_c.at[pl.ds((1 - par) * CH, CH)], sem_in)

      @plsc.parallel_loop(0, CH // 16, unroll=4)
      def prep(j):
        r = ebuf_r[pl.ds(par * CH + j * 16, 16)] - b * NB
        c = ebuf_c[pl.ds(par * CH + j * 16, 16)] - b * NB
        rp = plsc.load_gather(a1, [r])
        cp = plsc.load_gather(a1, [c])
        key = jnp.where(rp < 0, cp & 0x7FFFFFFF, SENT)
        cnt, is_last = plsc.scan_count(key)
        pkey[pl.ds(j * 16, 16)] = key
        pcnt[pl.ds(j * 16, 16)] = cnt
        pil[pl.ds(j * 16, 16)] = jnp.where(is_last, 1, 0)
        psc[pl.ds(j * 16, 16)] = plsc.load_gather(a3, [key])

      def row(jj, _2):
        rr = lax.rem(jj, NRING)

        def vreg(l, _3):
          j = jj * 8 + l
          key = pkey[pl.ds(j * 16, 16)]
          cnt = pcnt[pl.ds(j * 16, 16)]
          il = pil[pl.ds(j * 16, 16)] != 0
          base = plsc.load_gather(a2, [key])
          pos = base + cnt - 1
          plsc.store_scatter(a2, [key], base + cnt, mask=il)
          outpos = jnp.where(pos < E_K, pos, E_K + (pos & 511))
          stg_p[rr, pl.ds(l * 16, 16)] = g * OUTP + outpos
          stg_e[rr, pl.ds(l * 16, 16)] = base_e + j * 16 + iota
          stg_s[rr, pl.ds(l * 16, 16)] = psc[pl.ds(j * 16, 16)]
          return 0
        lax.fori_loop(0, 8, vreg, 0)
        pltpu.async_copy(stg_e.at[rr], oute_sh.at[stg_p.at[rr]], sem_out)
        pltpu.async_copy(stg_s.at[rr], outs_sh.at[stg_p.at[rr]], sem_out)

        @pl.when(jj >= 4)
        def _ringdrain():
          drain_row(lax.rem(jj - 4, NRING))
        return 0
      lax.fori_loop(0, CH // 128, row, 0)

      def tail_drain(jj, _2):
        drain_row(lax.rem(jj, NRING))
        return 0
      lax.fori_loop(CH // 128 - 4, CH // 128, tail_drain, 0)
      return 0
    lax.fori_loop(c0, c0 + nchunks, chunk, 0)

  plsc.subcore_barrier()

  @pl.when(t == 0)
  def _final():
    def out_chunk(k2, _):
      pltpu.sync_copy(oute_sh.at[pl.ds(g * OUTP + k2 * FCH, FCH)],
                      ebuf_r.at[pl.ds(0, FCH)])
      pltpu.sync_copy(ebuf_r.at[pl.ds(0, FCH)],
                      oute_hbm.at[pl.ds(b * E_K + k2 * FCH, FCH)])
      pltpu.sync_copy(outs_sh.at[pl.ds(g * OUTP + k2 * FCH, FCH)], buf_s)
      pltpu.sync_copy(buf_s, outs_hbm.at[pl.ds(b * E_K + k2 * FCH, FCH)])
      return 0
    lax.fori_loop(0, E_K // FCH, out_chunk, 0)


def kernel(score, edge_index, batch):
  mesh = plsc.VectorSubcoreMesh(core_axis_name="c", subcore_axis_name="s",
                                num_cores=2, num_subcores=16)
  f = pl.kernel(
      _body,
      compiler_params=pltpu.CompilerParams(needs_layout_passes=False),
      out_type=(jax.ShapeDtypeStruct((B * E_K,), jnp.int32),
                jax.ShapeDtypeStruct((B * E_K,), jnp.float32)),
      mesh=mesh,
      scratch_types=[
          pltpu.VMEM((NBP,), jnp.int32),
          pltpu.VMEM((NBP,), jnp.int32),
          pltpu.VMEM((NBP,), jnp.float32),
          pltpu.VMEM((2048,), jnp.int32),
          pltpu.VMEM((FCH,), jnp.float32),
          pltpu.VMEM((NRING, 128), jnp.int32),
          pltpu.VMEM((NRING, 128), jnp.int32),
          pltpu.VMEM((NRING, 128), jnp.float32),
          pltpu.VMEM((2 * CH,), jnp.int32),
          pltpu.VMEM((2 * CH,), jnp.int32),
          pltpu.VMEM((CH,), jnp.int32),
          pltpu.VMEM((CH,), jnp.int32),
          pltpu.VMEM((CH,), jnp.int32),
          pltpu.VMEM((CH,), jnp.float32),
          pltpu.SemaphoreType.DMA,
          pltpu.SemaphoreType.DMA,
          pltpu.VMEM_SHARED((2 * NBP,), jnp.int32),
          pltpu.VMEM_SHARED((2 * NBP,), jnp.int32),
          pltpu.VMEM_SHARED((14 * NBP,), jnp.int32),
          pltpu.VMEM_SHARED((2 * OUTP,), jnp.int32),
          pltpu.VMEM_SHARED((2 * OUTP,), jnp.float32),
      ],
  )
  return f(score, edge_index[0], edge_index[1], batch)


# speedup vs baseline: 507.3831x; 1.1434x over previous
import jax
import jax.numpy as jnp
from jax import lax
from jax.experimental import pallas as pl
from jax.experimental.pallas import tpu as pltpu
from jax.experimental.pallas import tpu_sc as plsc

N = 100000
B = 4
NB = 25000
DEG = 64
EB = NB * DEG
K_B = 2500
E_K = 40000

NPAD = 25008
NBP = 25024
SENT = 25008
CH = 1280
NCH = EB // CH
OUTP = 40960
NRING = 10
FCH = 800

_CHUNKS = [(i * 2000, 2000) for i in range(12)] + [(24000, 1024)]


def _key_from_score(s):
  bits = lax.bitcast_convert_type(s, jnp.uint32)
  neg = (bits >> jnp.uint32(31)) != jnp.uint32(0)
  u = jnp.where(neg, ~bits, bits | jnp.uint32(0x80000000))
  return ~u


def _node_key(score_ref, v):
  s = plsc.load_gather(score_ref, [v])
  k = _key_from_score(s)
  return jnp.where(v < NB, k, jnp.uint32(0xFFFFFFFF))


def _add_from_shared(dst, src_sh, src_base, buf):
  for off, sz in _CHUNKS:
    pltpu.sync_copy(src_sh.at[pl.ds(src_base + off, sz)],
                    buf.at[pl.ds(0, sz)])

    def addv(k2, _, off=off):
      dst[pl.ds(off + k2 * 16, 16)] = (
          dst[pl.ds(off + k2 * 16, 16)] + buf[pl.ds(k2 * 16, 16)])
      return 0

    lax.fori_loop(0, sz // 16, addv, 0)


def _zero(ref, nwords):
  def z(j, _):
    ref[pl.ds(j * 16, 16)] = jnp.zeros((16,), jnp.int32)
    return 0
  lax.fori_loop(0, nwords // 16, z, 0)


def _body(score_hbm, row_hbm, col_hbm, batch_hbm, oute_hbm, outs_hbm,
          a1, a2, a3, hist, buf_s, stg_p, stg_e, stg_s,
          ebuf_r, ebuf_c, pkey, pcnt, sem_in, sem_out,
          reppk_sh, wc_sh, mgrid, oute_sh, outs_sh):
  del batch_hbm
  cid = lax.axis_index("c")
  sid = lax.axis_index("s")
  g = sid // 8
  t = sid % 8
  b = cid * 2 + g
  iota = lax.iota(jnp.int32, 16)

  pltpu.sync_copy(score_hbm.at[pl.ds(b * NB, NB)], a3.at[pl.ds(0, NB)])

  @pl.when(t == 0)
  def _phase1():
    def init(j, _):
      a2[pl.ds(j * 16, 16)] = j * 16 + iota
      return 0
    lax.fori_loop(0, NPAD // 16, init, 0)

    for p in range(3):
      src, dst = (a2, a1) if p % 2 == 0 else (a1, a2)
      sh = jnp.uint32(11 * p)
      _zero(hist, 2048)

      @plsc.parallel_loop(0, NPAD // 16, unroll=4)
      def histo(j, src=src, sh=sh):
        v = src[pl.ds(j * 16, 16)]
        d = ((_node_key(a3, v) >> sh) & jnp.uint32(2047)).astype(jnp.int32)
        cnt, is_last = plsc.scan_count(d)
        plsc.addupdate_scatter(hist, [d], cnt, mask=is_last)

      def excl(h, carry):
        vv = hist[pl.ds(h * 16, 16)]
        inc = plsc.cumsum(vv)
        hist[pl.ds(h * 16, 16)] = inc - vv + carry
        return carry + jnp.max(inc)
      lax.fori_loop(0, 128, excl, jnp.int32(0))

      def qchunk(q, _, src=src, dst=dst, sh=sh):
        nv = jnp.minimum(80, NPAD // 16 - q * 80)

        @plsc.parallel_loop(0, nv, unroll=4)
        def prep2(jl):
          v = src[pl.ds((q * 80 + jl) * 16, 16)]
          d = ((_node_key(a3, v) >> sh) & jnp.uint32(2047)).astype(jnp.int32)
          cnt, is_last = plsc.scan_count(d)
          pkey[pl.ds(jl * 16, 16)] = d
          pcnt[pl.ds(jl * 16, 16)] = cnt | jnp.where(is_last,
                                                     jnp.int32(-2147483648),
                                                     jnp.int32(0))

        def place(jl, _2, src=src):
          d = pkey[pl.ds(jl * 16, 16)]
          raw = pcnt[pl.ds(jl * 16, 16)]
          cnt = raw & 0xFFFF
          il = raw < 0
          v = src[pl.ds((q * 80 + jl) * 16, 16)]
          base = plsc.load_gather(hist, [d])
          plsc.store_scatter(dst, [base + cnt - 1], v)
          plsc.store_scatter(hist, [d], base + cnt, mask=il)
          return 0
        lax.fori_loop(0, nv, place, 0)
        return 0
      lax.fori_loop(0, (NPAD // 16 + 79) // 80, qchunk, 0)

    def repb(j, carry_rep):
      v = a1[pl.ds(j * 16, 16)]
      vp = a1[pl.ds(jnp.maximum(j * 16 - 1, 0), 16)]
      k = _node_key(a3, v)
      kp = _node_key(a3, vp)
      neq = (k != kp) | (j == 0)
      packed = jnp.where(neq, iota * 32768 + v, -1)
      packed = jnp.where((iota == 0) & jnp.logical_not(neq), carry_rep, packed)
      pm = plsc.cummax(packed)
      rep = pm & 32767
      selbit = jnp.where((j * 16 + iota) < K_B, jnp.int32(-2147483648),
                         jnp.int32(0))
      plsc.store_scatter(a2, [v], rep | selbit)
      return jnp.max(pm) & 32767
    lax.fori_loop(0, NPAD // 16, repb, jnp.int32(0))
    pltpu.sync_copy(a2, reppk_sh.at[pl.ds(g * NBP, NBP)])

  plsc.subcore_barrier()

  c0 = (t - 1) * 178 + jnp.minimum(t - 1, 4)
  nchunks = jnp.where(t <= 4, 179, 178)

  @pl.when(t > 0)
  def _phase2():
    pltpu.sync_copy(reppk_sh.at[pl.ds(g * NBP, NBP)], a1)
    _zero(a2, NBP)

    pltpu.async_copy(row_hbm.at[pl.ds(b * EB + c0 * CH, CH)],
                     ebuf_r.at[pl.ds(0, CH)], sem_in)
    pltpu.async_copy(col_hbm.at[pl.ds(b * EB + c0 * CH, CH)],
                     ebuf_c.at[pl.ds(0, CH)], sem_in)

    def chunk(ci, _):
      par = lax.rem(ci - c0, 2)
      base_e = b * EB + ci * CH
      pltpu.make_async_copy(row_hbm.at[pl.ds(base_e, CH)],
                            ebuf_r.at[pl.ds(par * CH, CH)], sem_in).wait()
      pltpu.make_async_copy(col_hbm.at[pl.ds(base_e, CH)],
                            ebuf_c.at[pl.ds(par * CH, CH)], sem_in).wait()

      @pl.when(ci + 1 < c0 + nchunks)
      def _pref():
        nbase = b * EB + (ci + 1) * CH
        pltpu.async_copy(row_hbm.at[pl.ds(nbase, CH)],
                         ebuf_r.at[pl.ds((1 - par) * CH, CH)], sem_in)
        pltpu.async_copy(col_hbm.at[pl.ds(nbase, CH)],
                         ebuf_c.at[pl.ds((1 - par) * CH, CH)], sem_in)

      @plsc.parallel_loop(0, CH // 16, unroll=8)
      def vreg(j):
        r = ebuf_r[pl.ds(par * CH + j * 16, 16)] - b * NB
        c = ebuf_c[pl.ds(par * CH + j * 16, 16)] - b * NB
        rp = plsc.load_gather(a1, [r])
        cp = plsc.load_gather(a1, [c])
        key = jnp.where(rp < 0, cp & 0x7FFFFFFF, SENT)
        cnt, is_last = plsc.scan_count(key)
        plsc.addupdate_scatter(a2, [key], cnt, mask=is_last)
      return 0
    lax.fori_loop(c0, c0 + nchunks, chunk, 0)
    pltpu.sync_copy(a2, mgrid.at[pl.ds((g * 7 + t - 1) * NBP, NBP)])

  plsc.subcore_barrier()

  @pl.when(t == 0)
  def _phase3_lead():
    pltpu.sync_copy(mgrid.at[pl.ds(g * 7 * NBP, NBP)], a2)

    def acc_tile(t2, _):
      _add_from_shared(a2, mgrid, (g * 7 + t2 - 1) * NBP, ebuf_r)
      return 0
    lax.fori_loop(2, 8, acc_tile, 0)

    for l in range(8):
      stg_p[0, pl.ds(l * 16, 16)] = jnp.full((16,), g * NBP + SENT + 1,
                                             jnp.int32)

    def wrow(jj, carry):
      def wvreg(l, carry2):
        carry_cum, carry_w = carry2
        j = jj * 8 + l
        v = a1[pl.ds(j * 16, 16)]
        vp = a1[pl.ds(jnp.maximum(j * 16 - 1, 0), 16)]
        k = _node_key(a3, v)
        kp = _node_key(a3, vp)
        neq = (k != kp) | (j == 0)
        mv = plsc.load_gather(a2, [v])
        inc = plsc.cumsum(mv)
        excl2 = inc - mv + carry_cum
        w_in = jnp.where(neq, excl2, -1)
        w_in = jnp.where((iota == 0) & jnp.logical_not(neq), carry_w, w_in)
        wl = plsc.cummax(w_in)
        stg_p[0, pl.ds(l * 16, 16)] = g * NBP + v
        stg_e[0, pl.ds(l * 16, 16)] = wl
        return (carry_cum + jnp.max(inc), jnp.max(wl))
      nv = jnp.minimum(8, NPAD // 16 - jj * 8)
      carry = lax.fori_loop(0, nv, wvreg, carry)
      pltpu.sync_copy(stg_e.at[0], wc_sh.at[stg_p.at[0]])
      return carry
    lax.fori_loop(0, (NPAD // 16 + 7) // 8, wrow,
                  (jnp.int32(0), jnp.int32(0)))

    for l in range(8):
      stg_p[0, pl.ds(l * 16, 16)] = g * NBP + SENT + iota
      stg_e[0, pl.ds(l * 16, 16)] = jnp.full((16,), E_K, jnp.int32)
    pltpu.sync_copy(stg_e.at[0], wc_sh.at[stg_p.at[0]])

  @pl.when(t > 0)
  def _phase3a():
    _zero(a2, NBP)

    def acc_tile(t2, _):
      _add_from_shared(a2, mgrid, (g * 7 + t2 - 1) * NBP, ebuf_r)
      return 0
    lax.fori_loop(1, t, acc_tile, 0)

  plsc.subcore_barrier()

  @pl.when(t > 0)
  def _phase4():
    _add_from_shared(a2, wc_sh, g * NBP, ebuf_r)

    pltpu.async_copy(row_hbm.at[pl.ds(b * EB + c0 * CH, CH)],
                     ebuf_r.at[pl.ds(0, CH)], sem_in)
    pltpu.async_copy(col_hbm.at[pl.ds(b * EB + c0 * CH, CH)],
                     ebuf_c.at[pl.ds(0, CH)], sem_in)

    def drain_row(rr):
      pltpu.make_async_copy(stg_e.at[rr], oute_sh.at[stg_p.at[rr]],
                            sem_out).wait()
      pltpu.make_async_copy(stg_s.at[rr], outs_sh.at[stg_p.at[rr]],
                            sem_out).wait()

    def chunk(ci, _):
      par = lax.rem(ci - c0, 2)
      base_e = b * EB + ci * CH
      pltpu.make_async_copy(row_hbm.at[pl.ds(base_e, CH)],
                            ebuf_r.at[pl.ds(par * CH, CH)], sem_in).wait()
      pltpu.make_async_copy(col_hbm.at[pl.ds(base_e, CH)],
                            ebuf_c.at[pl.ds(par * CH, CH)], sem_in).wait()

      @pl.when(ci + 1 < c0 + nchunks)
      def _pref():
        nbase = b * EB + (ci + 1) * CH
        pltpu.async_copy(row_hbm.at[pl.ds(nbase, CH)],
                         ebuf_r.at[pl.ds((1 - par) * CH, CH)], sem_in)
        pltpu.async_copy(col_hbm.at[pl.ds(nbase, CH)],
                         ebuf_c.at[pl.ds((1 - par) * CH, CH)], sem_in)

      @plsc.parallel_loop(0, CH // 16, unroll=4)
      def prep(j):
        r = ebuf_r[pl.ds(par * CH + j * 16, 16)] - b * NB
        c = ebuf_c[pl.ds(par * CH + j * 16, 16)] - b * NB
        rp = plsc.load_gather(a1, [r])
        cp = plsc.load_gather(a1, [c])
        key = jnp.where(rp < 0, cp & 0x7FFFFFFF, SENT)
        cnt, is_last = plsc.scan_count(key)
        pkey[pl.ds(j * 16, 16)] = key
        pcnt[pl.ds(j * 16, 16)] = cnt | jnp.where(is_last,
                                                  jnp.int32(-2147483648),
                                                  jnp.int32(0))
        stg_e[j // 8, pl.ds((j % 8) * 16, 16)] = base_e + j * 16 + iota
        stg_s[j // 8, pl.ds((j % 8) * 16, 16)] = plsc.load_gather(a3, [key])

      def row(jj, _2):
        rr = jj

        def vreg(l, _3):
          j = jj * 8 + l
          key = pkey[pl.ds(j * 16, 16)]
          raw = pcnt[pl.ds(j * 16, 16)]
          cnt = raw & 0xFFFF
          il = raw < 0
          base = plsc.load_gather(a2, [key])
          pos = base + cnt - 1
          plsc.store_scatter(a2, [key], base + cnt, mask=il)
          outpos = jnp.where(pos < E_K, pos, E_K + (pos & 511))
          stg_p[rr, pl.ds(l * 16, 16)] = g * OUTP + outpos
          return 0
        lax.fori_loop(0, 8, vreg, 0)
        pltpu.async_copy(stg_e.at[rr], oute_sh.at[stg_p.at[rr]], sem_out)
        pltpu.async_copy(stg_s.at[rr], outs_sh.at[stg_p.at[rr]], sem_out)

        @pl.when(jj >= 4)
        def _ringdrain():
          drain_row(jj - 4)
        return 0
      lax.fori_loop(0, CH // 128, row, 0)

      def tail_drain(jj, _2):
        drain_row(jj)
        return 0
      lax.fori_loop(CH // 128 - 4, CH // 128, tail_drain, 0)
      return 0
    lax.fori_loop(c0, c0 + nchunks, chunk, 0)

  plsc.subcore_barrier()

  @pl.when(t == 0)
  def _final():
    def out_chunk(k2, _):
      pltpu.sync_copy(oute_sh.at[pl.ds(g * OUTP + k2 * FCH, FCH)],
                      ebuf_r.at[pl.ds(0, FCH)])
      pltpu.sync_copy(ebuf_r.at[pl.ds(0, FCH)],
                      oute_hbm.at[pl.ds(b * E_K + k2 * FCH, FCH)])
      pltpu.sync_copy(outs_sh.at[pl.ds(g * OUTP + k2 * FCH, FCH)], buf_s)
      pltpu.sync_copy(buf_s, outs_hbm.at[pl.ds(b * E_K + k2 * FCH, FCH)])
      return 0
    lax.fori_loop(0, E_K // FCH, out_chunk, 0)


def kernel(score, edge_index, batch):
  mesh = plsc.VectorSubcoreMesh(core_axis_name="c", subcore_axis_name="s",
                                num_cores=2, num_subcores=16)
  f = pl.kernel(
      _body,
      compiler_params=pltpu.CompilerParams(needs_layout_passes=False),
      out_type=(jax.ShapeDtypeStruct((B * E_K,), jnp.int32),
                jax.ShapeDtypeStruct((B * E_K,), jnp.float32)),
      mesh=mesh,
      scratch_types=[
          pltpu.VMEM((NBP,), jnp.int32),
          pltpu.VMEM((NBP,), jnp.int32),
          pltpu.VMEM((NBP,), jnp.float32),
          pltpu.VMEM((2048,), jnp.int32),
          pltpu.VMEM((FCH,), jnp.float32),
          pltpu.VMEM((NRING, 128), jnp.int32),
          pltpu.VMEM((NRING, 128), jnp.int32),
          pltpu.VMEM((NRING, 128), jnp.float32),
          pltpu.VMEM((2 * CH,), jnp.int32),
          pltpu.VMEM((2 * CH,), jnp.int32),
          pltpu.VMEM((CH,), jnp.int32),
          pltpu.VMEM((CH,), jnp.int32),
          pltpu.SemaphoreType.DMA,
          pltpu.SemaphoreType.DMA,
          pltpu.VMEM_SHARED((2 * NBP,), jnp.int32),
          pltpu.VMEM_SHARED((2 * NBP,), jnp.int32),
          pltpu.VMEM_SHARED((14 * NBP,), jnp.int32),
          pltpu.VMEM_SHARED((2 * OUTP,), jnp.int32),
          pltpu.VMEM_SHARED((2 * OUTP,), jnp.float32),
      ],
  )
  return f(score, edge_index[0], edge_index[1], batch)
